# Initial kernel scaffold; baseline (speedup 1.0000x reference)
import jax
import jax.numpy as jnp
from jax.experimental import pallas as pl


def kernel(x, edge_index, batch, W1_0, b1_0, a1_0, W2_0, b2_0, W1_1, b1_1, a1_1, W2_1, b2_1, a_out, g0, be0, g1, be1):
    raise NotImplementedError("write your pallas kernel here")


# speedup vs baseline: 5.5982x; 5.5982x over previous
import functools

import jax
import jax.numpy as jnp
from jax import lax
from jax.experimental import pallas as pl
from jax.experimental.pallas import tpu as pltpu
from jax.experimental.pallas import tpu_sc as plsc

_N, _D, _E = 10000, 128, 320000
_NC, _NS = 2, 16
_NW = _NC * _NS
_EPW = _E // _NW
_CH = 128
_NFULL = _EPW // _CH
_TAIL = _EPW - _NFULL * _CH
_NPAD = 10240
_ZR = _NPAD // _NS
_WR = 624
_WREM = _N - _NS * _WR

_mesh = plsc.VectorSubcoreMesh(
    core_axis_name="c", subcore_axis_name="s", num_cores=_NC, num_subcores=_NS)


@functools.partial(
    pl.kernel,
    out_type=jax.ShapeDtypeStruct((_NC, _N, _D), jnp.float32),
    mesh=_mesh,
    scratch_types=[
        pltpu.VMEM((_CH,), jnp.int32),
        pltpu.VMEM((_CH,), jnp.int32),
        pltpu.VMEM((_CH, _D), jnp.float32),
        pltpu.VMEM((_TAIL,), jnp.int32),
        pltpu.VMEM((_TAIL,), jnp.int32),
        pltpu.VMEM((_TAIL, _D), jnp.float32),
        pltpu.VMEM((128, _D), jnp.float32),
        pltpu.VMEM_SHARED((_NPAD, _D), jnp.float32),
        pltpu.SemaphoreType.DMA,
    ],
)
def _sc_scatter(z_hbm, src_hbm, dst_hbm, zeros_hbm, out_hbm,
                idx_s, idx_d, rows, idx_st, idx_dt, rows_t, zbuf, acc, sem):
    cid = lax.axis_index("c")
    sid = lax.axis_index("s")
    wid = cid * _NS + sid

    pltpu.sync_copy(zeros_hbm, zbuf)

    def zero_iter(k, _):
        pltpu.sync_copy(zbuf, acc.at[pl.ds(sid * _ZR + k * 128, 128)])
        return 0
    lax.fori_loop(0, _ZR // 128, zero_iter, 0)
    plsc.subcore_barrier()

    ebase = wid * _EPW

    def edge_iter(j, _):
        b = ebase + j * _CH
        pltpu.sync_copy(src_hbm.at[pl.ds(b, _CH)], idx_s)
        pltpu.sync_copy(dst_hbm.at[pl.ds(b, _CH)], idx_d)
        pltpu.async_copy(z_hbm.at[idx_s], rows, sem).wait()
        pltpu.sync_copy(rows, acc.at[idx_d], add=True)
        return 0
    lax.fori_loop(0, _NFULL, edge_iter, 0)

    bt = ebase + _NFULL * _CH
    pltpu.sync_copy(src_hbm.at[pl.ds(bt, _TAIL)], idx_st)
    pltpu.sync_copy(dst_hbm.at[pl.ds(bt, _TAIL)], idx_dt)
    pltpu.async_copy(z_hbm.at[idx_st], rows_t, sem).wait()
    pltpu.sync_copy(rows_t, acc.at[idx_dt], add=True)
    plsc.subcore_barrier()

    r0 = sid * _WR
    pltpu.sync_copy(acc.at[pl.ds(r0, _WR)], out_hbm.at[cid, pl.ds(r0, _WR)])

    @pl.when(sid == 0)
    def _():
        rr = _NS * _WR
        pltpu.sync_copy(acc.at[pl.ds(rr, _WREM)],
                        out_hbm.at[cid, pl.ds(rr, _WREM)])


_RB = 1000


def _mlp_body(z_ref, agg_ref, w1_ref, b1_ref, w2_ref, b2_ref, am_ref, ao_ref,
              p_ref, s_ref, q_ref):
    h = z_ref[...] + agg_ref[0] + agg_ref[1]
    h = jnp.dot(h, w1_ref[...], preferred_element_type=jnp.float32) + b1_ref[...]
    am = am_ref[0]
    h = jnp.where(h >= 0, h, am * h)
    y = jnp.dot(h, w2_ref[...], preferred_element_type=jnp.float32) + b2_ref[...]
    ao = ao_ref[0]
    p = jnp.where(y >= 0, y, ao * y)
    p_ref[...] = p
    ps = jnp.sum(p.reshape(_RB // 8, 8, _D), axis=0)
    pq = jnp.sum((p * p).reshape(_RB // 8, 8, _D), axis=0)

    @pl.when(pl.program_id(0) == 0)
    def _():
        s_ref[...] = jnp.zeros((8, _D), jnp.float32)
        q_ref[...] = jnp.zeros((8, _D), jnp.float32)

    s_ref[...] += ps
    q_ref[...] += pq


def _mlp(z, agg, W1, b1, W2, b2, am, ao):
    return pl.pallas_call(
        _mlp_body,
        grid=(_N // _RB,),
        in_specs=[
            pl.BlockSpec((_RB, _D), lambda i: (i, 0)),
            pl.BlockSpec((_NC, _RB, _D), lambda i: (0, i, 0)),
            pl.BlockSpec((_D, _D), lambda i: (0, 0)),
            pl.BlockSpec((1, _D), lambda i: (0, 0)),
            pl.BlockSpec((_D, _D), lambda i: (0, 0)),
            pl.BlockSpec((1, _D), lambda i: (0, 0)),
            pl.BlockSpec(memory_space=pltpu.SMEM),
            pl.BlockSpec(memory_space=pltpu.SMEM),
        ],
        out_specs=[
            pl.BlockSpec((_RB, _D), lambda i: (i, 0)),
            pl.BlockSpec((8, _D), lambda i: (0, 0)),
            pl.BlockSpec((8, _D), lambda i: (0, 0)),
        ],
        out_shape=[
            jax.ShapeDtypeStruct((_N, _D), jnp.float32),
            jax.ShapeDtypeStruct((8, _D), jnp.float32),
            jax.ShapeDtypeStruct((8, _D), jnp.float32),
        ],
    )(z, agg, W1, b1, W2, b2, am, ao)


def _bn_body(p_ref, s_ref, q_ref, g_ref, be_ref, o_ref):
    s = jnp.sum(s_ref[...], axis=0, keepdims=True)
    q = jnp.sum(q_ref[...], axis=0, keepdims=True)
    mu = s / _N
    var = q / _N - mu * mu
    inv = 1.0 / jnp.sqrt(var + 1e-5)
    o_ref[...] = g_ref[...] * (p_ref[...] - mu) * inv + be_ref[...]


def _bn(p, s, q, g, be):
    return pl.pallas_call(
        _bn_body,
        grid=(_N // _RB,),
        in_specs=[
            pl.BlockSpec((_RB, _D), lambda i: (i, 0)),
            pl.BlockSpec((8, _D), lambda i: (0, 0)),
            pl.BlockSpec((8, _D), lambda i: (0, 0)),
            pl.BlockSpec((1, _D), lambda i: (0, 0)),
            pl.BlockSpec((1, _D), lambda i: (0, 0)),
        ],
        out_specs=pl.BlockSpec((_RB, _D), lambda i: (i, 0)),
        out_shape=jax.ShapeDtypeStruct((_N, _D), jnp.float32),
    )(p, s, q, g, be)


def kernel(x, edge_index, batch, W1_0, b1_0, a1_0, W2_0, b2_0,
           W1_1, b1_1, a1_1, W2_1, b2_1, a_out, g0, be0, g1, be1):
    src = edge_index[0]
    dst = edge_index[1]
    zeros128 = jnp.zeros((128, _D), jnp.float32)
    ao = a_out.reshape(1)

    def layer(z, W1, b1, am, W2, b2, g, be):
        agg = _sc_scatter(z, src, dst, zeros128)
        p, s, q = _mlp(z, agg, W1, b1.reshape(1, _D), W2, b2.reshape(1, _D),
                       am.reshape(1), ao)
        return _bn(p, s, q, g.reshape(1, _D), be.reshape(1, _D))

    z = layer(x, W1_0, b1_0, a1_0, W2_0, b2_0, g0, be0)
    z = layer(z, W1_1, b1_1, a1_1, W2_1, b2_1, g1, be1)
    return z


# speedup vs baseline: 10.0270x; 1.7911x over previous
import functools

import jax
import jax.numpy as jnp
from jax import lax
from jax.experimental import pallas as pl
from jax.experimental.pallas import tpu as pltpu
from jax.experimental.pallas import tpu_sc as plsc

_N, _D, _E = 10000, 128, 320000
_NC, _NS = 2, 16
_NW = _NC * _NS
_EPW = _E // _NW
_CH = 80
_NFULL = _EPW // _CH
_WR = 624
_WREM = _N - _NS * _WR

_mesh = plsc.VectorSubcoreMesh(
    core_axis_name="c", subcore_axis_name="s", num_cores=_NC, num_subcores=_NS)


_NB = 2
_NG = (_NFULL - 1) // _NB


@functools.partial(
    pl.kernel,
    out_type=jax.ShapeDtypeStruct((_NC, _N, _D), jnp.float32),
    mesh=_mesh,
    scratch_types=[
        pltpu.VMEM((_EPW,), jnp.int32),
        pltpu.VMEM((_EPW,), jnp.int32),
        pltpu.VMEM((_CH,), jnp.int32),
        pltpu.VMEM((_CH,), jnp.int32),
        pltpu.VMEM((_CH, _D), jnp.float32),
        pltpu.VMEM((_CH, _D), jnp.float32),
        pltpu.VMEM_SHARED((_N, _D), jnp.float32),
        pltpu.SemaphoreType.DMA,
        pltpu.SemaphoreType.DMA,
    ],
)
def _sc_scatter(z_hbm, src_hbm, dst_hbm, zeros_hbm, out_hbm,
                src_all, dst_all, d0, d1, r0, r1, acc, s0, s1):
    dch = (d0, d1)
    rows = (r0, r1)
    gsem = (s0, s1)
    cid = lax.axis_index("c")
    sid = lax.axis_index("s")
    wid = cid * _NS + sid
    ebase = wid * _EPW

    pltpu.sync_copy(src_hbm.at[pl.ds(ebase, _EPW)], src_all)
    pltpu.sync_copy(dst_hbm.at[pl.ds(ebase, _EPW)], dst_all)

    pltpu.sync_copy(zeros_hbm, acc.at[pl.ds(sid * _WR, _WR)])

    @pl.when(sid == 0)
    def _():
        pltpu.sync_copy(zeros_hbm.at[pl.ds(0, _WREM)],
                        acc.at[pl.ds(_NS * _WR, _WREM)])
    plsc.subcore_barrier()

    def issue_gather(j, b):
        pltpu.async_copy(
            z_hbm.at[src_all.at[pl.ds(j * _CH, _CH)]], rows[b], gsem[b])

    def wait_gather(b):
        pltpu.make_async_copy(z_hbm.at[pl.ds(0, _CH)], rows[b], gsem[b]).wait()

    def fill_dst(j, b):
        for i in range(_CH // 16):
            dch[b][pl.ds(i * 16, 16)] = dst_all[pl.ds(j * _CH + i * 16, 16)]

    def step(j, b, issue_next):
        fill_dst(j, b)
        wait_gather(b)
        pltpu.sync_copy(rows[b], acc.at[dch[b]], add=True)
        if issue_next:
            issue_gather(j + _NB, b)

    for b in range(_NB):
        issue_gather(b, b)

    def group(g, _):
        for b in range(_NB):
            step(g * _NB + b, b, True)
        return 0
    lax.fori_loop(0, _NG - 1, group, 0)

    step((_NG - 1) * _NB, 0, True)
    step((_NG - 1) * _NB + 1, 1, False)
    step(_NG * _NB, 0, False)
    plsc.subcore_barrier()

    r0 = sid * _WR
    pltpu.sync_copy(acc.at[pl.ds(r0, _WR)], out_hbm.at[cid, pl.ds(r0, _WR)])

    @pl.when(sid == 0)
    def _():
        rr = _NS * _WR
        pltpu.sync_copy(acc.at[pl.ds(rr, _WREM)],
                        out_hbm.at[cid, pl.ds(rr, _WREM)])


_RB = 1000


def _mlp_body(z_ref, agg_ref, w1_ref, b1_ref, w2_ref, b2_ref, am_ref, ao_ref,
              p_ref, s_ref, q_ref):
    h = z_ref[...] + agg_ref[0] + agg_ref[1]
    h = jnp.dot(h, w1_ref[...], preferred_element_type=jnp.float32) + b1_ref[...]
    am = am_ref[0]
    h = jnp.where(h >= 0, h, am * h)
    y = jnp.dot(h, w2_ref[...], preferred_element_type=jnp.float32) + b2_ref[...]
    ao = ao_ref[0]
    p = jnp.where(y >= 0, y, ao * y)
    p_ref[...] = p
    ps = jnp.sum(p.reshape(_RB // 8, 8, _D), axis=0)
    pq = jnp.sum((p * p).reshape(_RB // 8, 8, _D), axis=0)

    @pl.when(pl.program_id(0) == 0)
    def _():
        s_ref[...] = jnp.zeros((8, _D), jnp.float32)
        q_ref[...] = jnp.zeros((8, _D), jnp.float32)

    s_ref[...] += ps
    q_ref[...] += pq


def _mlp(z, agg, W1, b1, W2, b2, am, ao):
    return pl.pallas_call(
        _mlp_body,
        grid=(_N // _RB,),
        in_specs=[
            pl.BlockSpec((_RB, _D), lambda i: (i, 0)),
            pl.BlockSpec((_NC, _RB, _D), lambda i: (0, i, 0)),
            pl.BlockSpec((_D, _D), lambda i: (0, 0)),
            pl.BlockSpec((1, _D), lambda i: (0, 0)),
            pl.BlockSpec((_D, _D), lambda i: (0, 0)),
            pl.BlockSpec((1, _D), lambda i: (0, 0)),
            pl.BlockSpec(memory_space=pltpu.SMEM),
            pl.BlockSpec(memory_space=pltpu.SMEM),
        ],
        out_specs=[
            pl.BlockSpec((_RB, _D), lambda i: (i, 0)),
            pl.BlockSpec((8, _D), lambda i: (0, 0)),
            pl.BlockSpec((8, _D), lambda i: (0, 0)),
        ],
        out_shape=[
            jax.ShapeDtypeStruct((_N, _D), jnp.float32),
            jax.ShapeDtypeStruct((8, _D), jnp.float32),
            jax.ShapeDtypeStruct((8, _D), jnp.float32),
        ],
    )(z, agg, W1, b1, W2, b2, am, ao)


def _bn_body(p_ref, s_ref, q_ref, g_ref, be_ref, o_ref):
    s = jnp.sum(s_ref[...], axis=0, keepdims=True)
    q = jnp.sum(q_ref[...], axis=0, keepdims=True)
    mu = s / _N
    var = q / _N - mu * mu
    inv = 1.0 / jnp.sqrt(var + 1e-5)
    o_ref[...] = g_ref[...] * (p_ref[...] - mu) * inv + be_ref[...]


def _bn(p, s, q, g, be):
    return pl.pallas_call(
        _bn_body,
        grid=(_N // _RB,),
        in_specs=[
            pl.BlockSpec((_RB, _D), lambda i: (i, 0)),
            pl.BlockSpec((8, _D), lambda i: (0, 0)),
            pl.BlockSpec((8, _D), lambda i: (0, 0)),
            pl.BlockSpec((1, _D), lambda i: (0, 0)),
            pl.BlockSpec((1, _D), lambda i: (0, 0)),
        ],
        out_specs=pl.BlockSpec((_RB, _D), lambda i: (i, 0)),
        out_shape=jax.ShapeDtypeStruct((_N, _D), jnp.float32),
    )(p, s, q, g, be)


def kernel(x, edge_index, batch, W1_0, b1_0, a1_0, W2_0, b2_0,
           W1_1, b1_1, a1_1, W2_1, b2_1, a_out, g0, be0, g1, be1):
    src = edge_index[0]
    dst = edge_index[1]
    zeros128 = jnp.zeros((_WR, _D), jnp.float32)
    ao = a_out.reshape(1)

    def layer(z, W1, b1, am, W2, b2, g, be):
        agg = _sc_scatter(z, src, dst, zeros128)
        p, s, q = _mlp(z, agg, W1, b1.reshape(1, _D), W2, b2.reshape(1, _D),
                       am.reshape(1), ao)
        return _bn(p, s, q, g.reshape(1, _D), be.reshape(1, _D))

    z = layer(x, W1_0, b1_0, a1_0, W2_0, b2_0, g0, be0)
    z = layer(z, W1_1, b1_1, a1_1, W2_1, b2_1, g1, be1)
    return z


# speedup vs baseline: 10.8132x; 1.0784x over previous
import functools

import jax
import jax.numpy as jnp
from jax import lax
from jax.experimental import pallas as pl
from jax.experimental.pallas import tpu as pltpu
from jax.experimental.pallas import tpu_sc as plsc

_N, _D, _E = 10000, 128, 320000
_NC, _NS = 2, 16
_NW = _NC * _NS
_EPW = _E // _NW
_CH = 40
_NFULL = _EPW // _CH
_WR = 624
_WREM = _N - _NS * _WR

_mesh = plsc.VectorSubcoreMesh(
    core_axis_name="c", subcore_axis_name="s", num_cores=_NC, num_subcores=_NS)


_NB = 4
_NG = (_NFULL - _NB - 2) // _NB


@functools.partial(
    pl.kernel,
    out_type=jax.ShapeDtypeStruct((_NC, _N, _D), jnp.float32),
    mesh=_mesh,
    scratch_types=[
        pltpu.VMEM((_EPW,), jnp.int32),
        pltpu.VMEM((_EPW,), jnp.int32),
        pltpu.VMEM((_CH,), jnp.int32),
        pltpu.VMEM((_CH,), jnp.int32),
        pltpu.VMEM((_CH,), jnp.int32),
        pltpu.VMEM((_CH,), jnp.int32),
        pltpu.VMEM((_CH, _D), jnp.float32),
        pltpu.VMEM((_CH, _D), jnp.float32),
        pltpu.VMEM((_CH, _D), jnp.float32),
        pltpu.VMEM((_CH, _D), jnp.float32),
        pltpu.VMEM_SHARED((_N, _D), jnp.float32),
        pltpu.SemaphoreType.DMA,
        pltpu.SemaphoreType.DMA,
        pltpu.SemaphoreType.DMA,
        pltpu.SemaphoreType.DMA,
        pltpu.SemaphoreType.DMA,
        pltpu.SemaphoreType.DMA,
        pltpu.SemaphoreType.DMA,
        pltpu.SemaphoreType.DMA,
    ],
)
def _sc_scatter(z_hbm, src_hbm, dst_hbm, zeros_hbm, out_hbm,
                src_all, dst_all, d0, d1, d2, d3, r0, r1, r2, r3, acc,
                g0, g1, g2, g3, c0, c1, c2, c3):
    dch = (d0, d1, d2, d3)
    rows = (r0, r1, r2, r3)
    gsem = (g0, g1, g2, g3)
    ssem = (c0, c1, c2, c3)
    cid = lax.axis_index("c")
    sid = lax.axis_index("s")
    wid = cid * _NS + sid
    ebase = wid * _EPW

    pltpu.sync_copy(src_hbm.at[pl.ds(ebase, _EPW)], src_all)
    pltpu.sync_copy(dst_hbm.at[pl.ds(ebase, _EPW)], dst_all)

    pltpu.sync_copy(zeros_hbm, acc.at[pl.ds(sid * _WR, _WR)])

    @pl.when(sid == 0)
    def _():
        pltpu.sync_copy(zeros_hbm.at[pl.ds(0, _WREM)],
                        acc.at[pl.ds(_NS * _WR, _WREM)])
    plsc.subcore_barrier()

    def issue_gather(j, b):
        pltpu.async_copy(
            z_hbm.at[src_all.at[pl.ds(j * _CH, _CH)]], rows[b], gsem[b])

    def wait_gather(b):
        pltpu.make_async_copy(z_hbm.at[pl.ds(0, _CH)], rows[b], gsem[b]).wait()

    def issue_scatter(b):
        pltpu.async_copy(rows[b], acc.at[dch[b]], ssem[b], add=True)

    def wait_scatter(b):
        pltpu.make_async_copy(rows[b], acc.at[dch[b]], ssem[b]).wait()

    def fill_dst(j, b):
        dch[b][pl.ds(0, 16)] = dst_all[pl.ds(j * _CH, 16)]
        dch[b][pl.ds(16, 16)] = dst_all[pl.ds(j * _CH + 16, 16)]
        dch[b][pl.ds(24, 16)] = dst_all[pl.ds(j * _CH + 24, 16)]

    def pipe_step(j, b, do_wait, do_issue):
        sslot = (b + 2) % _NB
        if do_wait:
            wait_scatter(sslot)
        if do_issue:
            issue_gather(j + 2, sslot)
        fill_dst(j, b)
        wait_gather(b)
        issue_scatter(b)

    issue_gather(0, 0)
    issue_gather(1, 1)
    pipe_step(0, 0, False, True)
    pipe_step(1, 1, False, True)
    pipe_step(2, 2, True, True)
    pipe_step(3, 3, True, True)

    def group(g, _):
        for b in range(_NB):
            pipe_step(g * _NB + b, b, True, True)
        return 0
    lax.fori_loop(1, _NG + 1, group, 0)

    pipe_step(_NFULL - 2, (_NFULL - 2) % _NB, True, False)
    pipe_step(_NFULL - 1, (_NFULL - 1) % _NB, True, False)
    wait_scatter((_NFULL - 2) % _NB)
    wait_scatter((_NFULL - 1) % _NB)
    plsc.subcore_barrier()

    r0 = sid * _WR
    pltpu.sync_copy(acc.at[pl.ds(r0, _WR)], out_hbm.at[cid, pl.ds(r0, _WR)])

    @pl.when(sid == 0)
    def _():
        rr = _NS * _WR
        pltpu.sync_copy(acc.at[pl.ds(rr, _WREM)],
                        out_hbm.at[cid, pl.ds(rr, _WREM)])


_RB = 1000


def _mlp_body(z_ref, agg_ref, w1_ref, b1_ref, w2_ref, b2_ref, am_ref, ao_ref,
              p_ref, s_ref, q_ref):
    h = z_ref[...] + agg_ref[0] + agg_ref[1]
    h = jnp.dot(h, w1_ref[...], preferred_element_type=jnp.float32) + b1_ref[...]
    am = am_ref[0]
    h = jnp.where(h >= 0, h, am * h)
    y = jnp.dot(h, w2_ref[...], preferred_element_type=jnp.float32) + b2_ref[...]
    ao = ao_ref[0]
    p = jnp.where(y >= 0, y, ao * y)
    p_ref[...] = p
    ps = jnp.sum(p.reshape(_RB // 8, 8, _D), axis=0)
    pq = jnp.sum((p * p).reshape(_RB // 8, 8, _D), axis=0)

    @pl.when(pl.program_id(0) == 0)
    def _():
        s_ref[...] = jnp.zeros((8, _D), jnp.float32)
        q_ref[...] = jnp.zeros((8, _D), jnp.float32)

    s_ref[...] += ps
    q_ref[...] += pq


def _mlp(z, agg, W1, b1, W2, b2, am, ao):
    return pl.pallas_call(
        _mlp_body,
        grid=(_N // _RB,),
        in_specs=[
            pl.BlockSpec((_RB, _D), lambda i: (i, 0)),
            pl.BlockSpec((_NC, _RB, _D), lambda i: (0, i, 0)),
            pl.BlockSpec((_D, _D), lambda i: (0, 0)),
            pl.BlockSpec((1, _D), lambda i: (0, 0)),
            pl.BlockSpec((_D, _D), lambda i: (0, 0)),
            pl.BlockSpec((1, _D), lambda i: (0, 0)),
            pl.BlockSpec(memory_space=pltpu.SMEM),
            pl.BlockSpec(memory_space=pltpu.SMEM),
        ],
        out_specs=[
            pl.BlockSpec((_RB, _D), lambda i: (i, 0)),
            pl.BlockSpec((8, _D), lambda i: (0, 0)),
            pl.BlockSpec((8, _D), lambda i: (0, 0)),
        ],
        out_shape=[
            jax.ShapeDtypeStruct((_N, _D), jnp.float32),
            jax.ShapeDtypeStruct((8, _D), jnp.float32),
            jax.ShapeDtypeStruct((8, _D), jnp.float32),
        ],
    )(z, agg, W1, b1, W2, b2, am, ao)


def _bn_body(p_ref, s_ref, q_ref, g_ref, be_ref, o_ref):
    s = jnp.sum(s_ref[...], axis=0, keepdims=True)
    q = jnp.sum(q_ref[...], axis=0, keepdims=True)
    mu = s / _N
    var = q / _N - mu * mu
    inv = 1.0 / jnp.sqrt(var + 1e-5)
    o_ref[...] = g_ref[...] * (p_ref[...] - mu) * inv + be_ref[...]


def _bn(p, s, q, g, be):
    return pl.pallas_call(
        _bn_body,
        grid=(_N // _RB,),
        in_specs=[
            pl.BlockSpec((_RB, _D), lambda i: (i, 0)),
            pl.BlockSpec((8, _D), lambda i: (0, 0)),
            pl.BlockSpec((8, _D), lambda i: (0, 0)),
            pl.BlockSpec((1, _D), lambda i: (0, 0)),
            pl.BlockSpec((1, _D), lambda i: (0, 0)),
        ],
        out_specs=pl.BlockSpec((_RB, _D), lambda i: (i, 0)),
        out_shape=jax.ShapeDtypeStruct((_N, _D), jnp.float32),
    )(p, s, q, g, be)


def kernel(x, edge_index, batch, W1_0, b1_0, a1_0, W2_0, b2_0,
           W1_1, b1_1, a1_1, W2_1, b2_1, a_out, g0, be0, g1, be1):
    src = edge_index[0]
    dst = edge_index[1]
    zeros128 = jnp.zeros((_WR, _D), jnp.float32)
    ao = a_out.reshape(1)

    def layer(z, W1, b1, am, W2, b2, g, be):
        agg = _sc_scatter(z, src, dst, zeros128)
        p, s, q = _mlp(z, agg, W1, b1.reshape(1, _D), W2, b2.reshape(1, _D),
                       am.reshape(1), ao)
        return _bn(p, s, q, g.reshape(1, _D), be.reshape(1, _D))

    z = layer(x, W1_0, b1_0, a1_0, W2_0, b2_0, g0, be0)
    z = layer(z, W1_1, b1_1, a1_1, W2_1, b2_1, g1, be1)
    return z


# speedup vs baseline: 11.2995x; 1.0450x over previous
import functools

import jax
import jax.numpy as jnp
from jax import lax
from jax.experimental import pallas as pl
from jax.experimental.pallas import tpu as pltpu
from jax.experimental.pallas import tpu_sc as plsc

_N, _D, _E = 10000, 128, 320000
_NC, _NS = 2, 16
_NW = _NC * _NS
_EPW = _E // _NW
_CH = 40
_NFULL = _EPW // _CH
_WR = 624
_WREM = _N - _NS * _WR

_mesh = plsc.VectorSubcoreMesh(
    core_axis_name="c", subcore_axis_name="s", num_cores=_NC, num_subcores=_NS)


_NB = 4
_NG = (_NFULL - _NB - 2) // _NB


@functools.partial(
    pl.kernel,
    out_type=jax.ShapeDtypeStruct((_NC, _N, _D), jnp.float32),
    mesh=_mesh,
    scratch_types=[
        pltpu.VMEM((_EPW,), jnp.int32),
        pltpu.VMEM((_EPW,), jnp.int32),
        pltpu.VMEM((_CH,), jnp.int32),
        pltpu.VMEM((_CH,), jnp.int32),
        pltpu.VMEM((_CH,), jnp.int32),
        pltpu.VMEM((_CH,), jnp.int32),
        pltpu.VMEM((_CH, _D), jnp.float32),
        pltpu.VMEM((_CH, _D), jnp.float32),
        pltpu.VMEM((_CH, _D), jnp.float32),
        pltpu.VMEM((_CH, _D), jnp.float32),
        pltpu.VMEM_SHARED((_N, _D), jnp.float32),
        pltpu.SemaphoreType.DMA,
        pltpu.SemaphoreType.DMA,
        pltpu.SemaphoreType.DMA,
        pltpu.SemaphoreType.DMA,
        pltpu.SemaphoreType.DMA,
        pltpu.SemaphoreType.DMA,
        pltpu.SemaphoreType.DMA,
        pltpu.SemaphoreType.DMA,
    ],
)
def _sc_scatter(z_hbm, src_hbm, dst_hbm, zeros_hbm, out_hbm,
                src_all, dst_all, d0, d1, d2, d3, r0, r1, r2, r3, acc,
                g0, g1, g2, g3, c0, c1, c2, c3):
    dch = (d0, d1, d2, d3)
    rows = (r0, r1, r2, r3)
    gsem = (g0, g1, g2, g3)
    ssem = (c0, c1, c2, c3)
    cid = lax.axis_index("c")
    sid = lax.axis_index("s")
    wid = cid * _NS + sid
    ebase = wid * _EPW

    pltpu.sync_copy(src_hbm.at[pl.ds(ebase, _EPW)], src_all)
    pltpu.sync_copy(dst_hbm.at[pl.ds(ebase, _EPW)], dst_all)

    pltpu.sync_copy(zeros_hbm, acc.at[pl.ds(sid * _WR, _WR)])

    @pl.when(sid == 0)
    def _():
        pltpu.sync_copy(zeros_hbm.at[pl.ds(0, _WREM)],
                        acc.at[pl.ds(_NS * _WR, _WREM)])
    plsc.subcore_barrier()

    def issue_gather(j, b):
        pltpu.async_copy(
            z_hbm.at[src_all.at[pl.ds(j * _CH, _CH)]], rows[b], gsem[b])

    def wait_gather(b):
        pltpu.make_async_copy(z_hbm.at[pl.ds(0, _CH)], rows[b], gsem[b]).wait()

    def issue_scatter(b):
        pltpu.async_copy(rows[b], acc.at[dch[b]], ssem[b], add=True)

    def wait_scatter(b):
        pltpu.make_async_copy(rows[b], acc.at[dch[b]], ssem[b]).wait()

    def fill_dst(j, b):
        dch[b][pl.ds(0, 16)] = dst_all[pl.ds(j * _CH, 16)]
        dch[b][pl.ds(16, 16)] = dst_all[pl.ds(j * _CH + 16, 16)]
        dch[b][pl.ds(24, 16)] = dst_all[pl.ds(j * _CH + 24, 16)]

    def pipe_step(j, b, do_wait, do_issue):
        sslot = (b + 2) % _NB
        if do_wait:
            wait_scatter(sslot)
        if do_issue:
            issue_gather(j + 2, sslot)
        fill_dst(j, b)
        wait_gather(b)
        issue_scatter(b)

    issue_gather(0, 0)
    issue_gather(1, 1)
    pipe_step(0, 0, False, True)
    pipe_step(1, 1, False, True)
    pipe_step(2, 2, True, True)
    pipe_step(3, 3, True, True)

    def group(g, _):
        for b in range(_NB):
            pipe_step(g * _NB + b, b, True, True)
        return 0
    lax.fori_loop(1, _NG + 1, group, 0)

    pipe_step(_NFULL - 2, (_NFULL - 2) % _NB, True, False)
    pipe_step(_NFULL - 1, (_NFULL - 1) % _NB, True, False)
    wait_scatter((_NFULL - 2) % _NB)
    wait_scatter((_NFULL - 1) % _NB)
    plsc.subcore_barrier()

    r0 = sid * _WR
    pltpu.sync_copy(acc.at[pl.ds(r0, _WR)], out_hbm.at[cid, pl.ds(r0, _WR)])

    @pl.when(sid == 0)
    def _():
        rr = _NS * _WR
        pltpu.sync_copy(acc.at[pl.ds(rr, _WREM)],
                        out_hbm.at[cid, pl.ds(rr, _WREM)])


_RB = 1000
_NGRID = _N // _RB


def _layer_body(z_ref, agg_ref, w1_ref, b1_ref, w2_ref, b2_ref, g_ref, be_ref,
                am_ref, ao_ref, o_ref, p_scr, s_scr, q_scr):
    i = pl.program_id(0)

    @pl.when(i < _NGRID)
    def _():
        h = z_ref[...] + agg_ref[0] + agg_ref[1]
        h = (jnp.dot(h, w1_ref[...], preferred_element_type=jnp.float32)
             + b1_ref[...])
        am = am_ref[0]
        h = jnp.where(h >= 0, h, am * h)
        y = (jnp.dot(h, w2_ref[...], preferred_element_type=jnp.float32)
             + b2_ref[...])
        ao = ao_ref[0]
        p = jnp.where(y >= 0, y, ao * y)
        p_scr[pl.ds(i * _RB, _RB), :] = p
        ps = jnp.sum(p.reshape(_RB // 8, 8, _D), axis=0)
        pq = jnp.sum((p * p).reshape(_RB // 8, 8, _D), axis=0)

        @pl.when(i == 0)
        def _():
            s_scr[...] = jnp.zeros((8, _D), jnp.float32)
            q_scr[...] = jnp.zeros((8, _D), jnp.float32)

        s_scr[...] += ps
        q_scr[...] += pq

    @pl.when(i >= _NGRID)
    def _():
        k = i - _NGRID
        s = jnp.sum(s_scr[...], axis=0, keepdims=True)
        q = jnp.sum(q_scr[...], axis=0, keepdims=True)
        mu = s / _N
        var = q / _N - mu * mu
        inv = 1.0 / jnp.sqrt(var + 1e-5)
        p = p_scr[pl.ds(k * _RB, _RB), :]
        o_ref[...] = g_ref[...] * (p - mu) * inv + be_ref[...]


def _layer_tc(z, agg, W1, b1, W2, b2, g, be, am, ao):
    clamp = lambda i: (jnp.minimum(i, _NGRID - 1), 0)
    fixed = lambda i: (0, 0)
    return pl.pallas_call(
        _layer_body,
        grid=(2 * _NGRID,),
        in_specs=[
            pl.BlockSpec((_RB, _D), clamp),
            pl.BlockSpec((_NC, _RB, _D), lambda i: (0, jnp.minimum(i, _NGRID - 1), 0)),
            pl.BlockSpec((_D, _D), fixed),
            pl.BlockSpec((1, _D), fixed),
            pl.BlockSpec((_D, _D), fixed),
            pl.BlockSpec((1, _D), fixed),
            pl.BlockSpec((1, _D), fixed),
            pl.BlockSpec((1, _D), fixed),
            pl.BlockSpec(memory_space=pltpu.SMEM),
            pl.BlockSpec(memory_space=pltpu.SMEM),
        ],
        out_specs=pl.BlockSpec(
            (_RB, _D), lambda i: (jnp.where(i < _NGRID, 0, i - _NGRID), 0)),
        out_shape=jax.ShapeDtypeStruct((_N, _D), jnp.float32),
        scratch_shapes=[
            pltpu.VMEM((_N, _D), jnp.float32),
            pltpu.VMEM((8, _D), jnp.float32),
            pltpu.VMEM((8, _D), jnp.float32),
        ],
    )(z, agg, W1, b1, W2, b2, g, be, am, ao)


def kernel(x, edge_index, batch, W1_0, b1_0, a1_0, W2_0, b2_0,
           W1_1, b1_1, a1_1, W2_1, b2_1, a_out, g0, be0, g1, be1):
    src = edge_index[0]
    dst = edge_index[1]
    zeros128 = jnp.zeros((_WR, _D), jnp.float32)
    ao = a_out.reshape(1)

    def layer(z, W1, b1, am, W2, b2, g, be):
        agg = _sc_scatter(z, src, dst, zeros128)
        return _layer_tc(z, agg, W1, b1.reshape(1, _D), W2, b2.reshape(1, _D),
                         g.reshape(1, _D), be.reshape(1, _D), am.reshape(1), ao)

    z = layer(x, W1_0, b1_0, a1_0, W2_0, b2_0, g0, be0)
    z = layer(z, W1_1, b1_1, a1_1, W2_1, b2_1, g1, be1)
    return z


# speedup vs baseline: 11.4323x; 1.0117x over previous
import functools

import jax
import jax.numpy as jnp
from jax import lax
from jax.experimental import pallas as pl
from jax.experimental.pallas import tpu as pltpu
from jax.experimental.pallas import tpu_sc as plsc

_N, _D, _E = 10000, 128, 320000
_NC, _NS = 2, 16
_NW = _NC * _NS
_EPW = _E // _NW
_CH = 40
_NFULL = _EPW // _CH
_WR = 624
_WREM = _N - _NS * _WR

_mesh = plsc.VectorSubcoreMesh(
    core_axis_name="c", subcore_axis_name="s", num_cores=_NC, num_subcores=_NS)


_NB = 4
_NG = (_NFULL - _NB - 2) // _NB


@functools.partial(
    pl.kernel,
    out_type=jax.ShapeDtypeStruct((_NC, _N, _D), jnp.float32),
    mesh=_mesh,
    scratch_types=[
        pltpu.VMEM((_EPW,), jnp.int32),
        pltpu.VMEM((_EPW,), jnp.int32),
        pltpu.VMEM((_CH,), jnp.int32),
        pltpu.VMEM((_CH,), jnp.int32),
        pltpu.VMEM((_CH,), jnp.int32),
        pltpu.VMEM((_CH,), jnp.int32),
        pltpu.VMEM((_CH, _D), jnp.float32),
        pltpu.VMEM((_CH, _D), jnp.float32),
        pltpu.VMEM((_CH, _D), jnp.float32),
        pltpu.VMEM((_CH, _D), jnp.float32),
        pltpu.VMEM_SHARED((_N, _D), jnp.float32),
        pltpu.SemaphoreType.DMA,
        pltpu.SemaphoreType.DMA,
        pltpu.SemaphoreType.DMA,
        pltpu.SemaphoreType.DMA,
        pltpu.SemaphoreType.DMA,
        pltpu.SemaphoreType.DMA,
        pltpu.SemaphoreType.DMA,
        pltpu.SemaphoreType.DMA,
    ],
)
def _sc_scatter(z_hbm, src_hbm, dst_hbm, zeros_hbm, out_hbm,
                src_all, dst_all, d0, d1, d2, d3, r0, r1, r2, r3, acc,
                g0, g1, g2, g3, c0, c1, c2, c3):
    dch = (d0, d1, d2, d3)
    rows = (r0, r1, r2, r3)
    gsem = (g0, g1, g2, g3)
    ssem = (c0, c1, c2, c3)
    cid = lax.axis_index("c")
    sid = lax.axis_index("s")
    wid = cid * _NS + sid
    ebase = wid * _EPW

    pltpu.async_copy(src_hbm.at[pl.ds(ebase, _EPW)], src_all, g0)
    pltpu.async_copy(dst_hbm.at[pl.ds(ebase, _EPW)], dst_all, g1)
    pltpu.async_copy(zeros_hbm, acc.at[pl.ds(sid * _WR, _WR)], g2)

    @pl.when(sid == 0)
    def _():
        pltpu.sync_copy(zeros_hbm.at[pl.ds(0, _WREM)],
                        acc.at[pl.ds(_NS * _WR, _WREM)])
    pltpu.make_async_copy(src_hbm.at[pl.ds(0, _EPW)], src_all, g0).wait()
    pltpu.make_async_copy(dst_hbm.at[pl.ds(0, _EPW)], dst_all, g1).wait()
    pltpu.make_async_copy(zeros_hbm, acc.at[pl.ds(0, _WR)], g2).wait()
    plsc.subcore_barrier()

    def issue_gather(j, b):
        pltpu.async_copy(
            z_hbm.at[src_all.at[pl.ds(j * _CH, _CH)]], rows[b], gsem[b])

    def wait_gather(b):
        pltpu.make_async_copy(z_hbm.at[pl.ds(0, _CH)], rows[b], gsem[b]).wait()

    def issue_scatter(b):
        pltpu.async_copy(rows[b], acc.at[dch[b]], ssem[b], add=True)

    def wait_scatter(b):
        pltpu.make_async_copy(rows[b], acc.at[dch[b]], ssem[b]).wait()

    def fill_dst(j, b):
        dch[b][pl.ds(0, 16)] = dst_all[pl.ds(j * _CH, 16)]
        dch[b][pl.ds(16, 16)] = dst_all[pl.ds(j * _CH + 16, 16)]
        dch[b][pl.ds(24, 16)] = dst_all[pl.ds(j * _CH + 24, 16)]

    def pipe_step(j, b, do_wait, do_issue):
        sslot = (b + 2) % _NB
        if do_wait:
            wait_scatter(sslot)
        if do_issue:
            issue_gather(j + 2, sslot)
        fill_dst(j, b)
        wait_gather(b)
        issue_scatter(b)

    issue_gather(0, 0)
    issue_gather(1, 1)
    pipe_step(0, 0, False, True)
    pipe_step(1, 1, False, True)
    pipe_step(2, 2, True, True)
    pipe_step(3, 3, True, True)

    def group(g, _):
        for b in range(_NB):
            pipe_step(g * _NB + b, b, True, True)
        return 0
    lax.fori_loop(1, _NG + 1, group, 0)

    pipe_step(_NFULL - 2, (_NFULL - 2) % _NB, True, False)
    pipe_step(_NFULL - 1, (_NFULL - 1) % _NB, True, False)
    wait_scatter((_NFULL - 2) % _NB)
    wait_scatter((_NFULL - 1) % _NB)
    plsc.subcore_barrier()

    r0 = sid * _WR
    pltpu.sync_copy(acc.at[pl.ds(r0, _WR)], out_hbm.at[cid, pl.ds(r0, _WR)])

    @pl.when(sid == 0)
    def _():
        rr = _NS * _WR
        pltpu.sync_copy(acc.at[pl.ds(rr, _WREM)],
                        out_hbm.at[cid, pl.ds(rr, _WREM)])


_RB = 1000
_NGRID = _N // _RB


def _layer_body(z_ref, agg_ref, w1_ref, b1_ref, w2_ref, b2_ref, g_ref, be_ref,
                am_ref, ao_ref, o_ref, p_scr, s_scr, q_scr):
    i = pl.program_id(0)

    @pl.when(i < _NGRID)
    def _():
        h = z_ref[...] + agg_ref[0] + agg_ref[1]
        h = (jnp.dot(h, w1_ref[...], preferred_element_type=jnp.float32)
             + b1_ref[...])
        am = am_ref[0]
        h = jnp.where(h >= 0, h, am * h)
        y = (jnp.dot(h, w2_ref[...], preferred_element_type=jnp.float32)
             + b2_ref[...])
        ao = ao_ref[0]
        p = jnp.where(y >= 0, y, ao * y)
        p_scr[pl.ds(i * _RB, _RB), :] = p
        ps = jnp.sum(p.reshape(_RB // 8, 8, _D), axis=0)
        pq = jnp.sum((p * p).reshape(_RB // 8, 8, _D), axis=0)

        @pl.when(i == 0)
        def _():
            s_scr[...] = jnp.zeros((8, _D), jnp.float32)
            q_scr[...] = jnp.zeros((8, _D), jnp.float32)

        s_scr[...] += ps
        q_scr[...] += pq

    @pl.when(i >= _NGRID)
    def _():
        k = i - _NGRID
        s = jnp.sum(s_scr[...], axis=0, keepdims=True)
        q = jnp.sum(q_scr[...], axis=0, keepdims=True)
        mu = s / _N
        var = q / _N - mu * mu
        inv = 1.0 / jnp.sqrt(var + 1e-5)
        p = p_scr[pl.ds(k * _RB, _RB), :]
        o_ref[...] = g_ref[...] * (p - mu) * inv + be_ref[...]


def _layer_tc(z, agg, W1, b1, W2, b2, g, be, am, ao):
    clamp = lambda i: (jnp.minimum(i, _NGRID - 1), 0)
    fixed = lambda i: (0, 0)
    return pl.pallas_call(
        _layer_body,
        grid=(2 * _NGRID,),
        in_specs=[
            pl.BlockSpec((_RB, _D), clamp),
            pl.BlockSpec((_NC, _RB, _D), lambda i: (0, jnp.minimum(i, _NGRID - 1), 0)),
            pl.BlockSpec((_D, _D), fixed),
            pl.BlockSpec((1, _D), fixed),
            pl.BlockSpec((_D, _D), fixed),
            pl.BlockSpec((1, _D), fixed),
            pl.BlockSpec((1, _D), fixed),
            pl.BlockSpec((1, _D), fixed),
            pl.BlockSpec(memory_space=pltpu.SMEM),
            pl.BlockSpec(memory_space=pltpu.SMEM),
        ],
        out_specs=pl.BlockSpec(
            (_RB, _D), lambda i: (jnp.where(i < _NGRID, 0, i - _NGRID), 0)),
        out_shape=jax.ShapeDtypeStruct((_N, _D), jnp.float32),
        scratch_shapes=[
            pltpu.VMEM((_N, _D), jnp.float32),
            pltpu.VMEM((8, _D), jnp.float32),
            pltpu.VMEM((8, _D), jnp.float32),
        ],
    )(z, agg, W1, b1, W2, b2, g, be, am, ao)


def kernel(x, edge_index, batch, W1_0, b1_0, a1_0, W2_0, b2_0,
           W1_1, b1_1, a1_1, W2_1, b2_1, a_out, g0, be0, g1, be1):
    src = edge_index[0]
    dst = edge_index[1]
    zeros128 = jnp.zeros((_WR, _D), jnp.float32)
    ao = a_out.reshape(1)

    def layer(z, W1, b1, am, W2, b2, g, be):
        agg = _sc_scatter(z, src, dst, zeros128)
        return _layer_tc(z, agg, W1, b1.reshape(1, _D), W2, b2.reshape(1, _D),
                         g.reshape(1, _D), be.reshape(1, _D), am.reshape(1), ao)

    z = layer(x, W1_0, b1_0, a1_0, W2_0, b2_0, g0, be0)
    z = layer(z, W1_1, b1_1, a1_1, W2_1, b2_1, g1, be1)
    return z


# speedup vs baseline: 12.1886x; 1.0662x over previous
import functools

import jax
import jax.numpy as jnp
from jax import lax
from jax.experimental import pallas as pl
from jax.experimental.pallas import tpu as pltpu
from jax.experimental.pallas import tpu_sc as plsc

_N, _D, _E = 10000, 128, 320000
_NC, _NS = 2, 16
_NW = _NC * _NS
_EPW = _E // _NW
_CH = 40
_NFULL = _EPW // _CH
_WR = 624
_WREM = _N - _NS * _WR

_mesh = plsc.VectorSubcoreMesh(
    core_axis_name="c", subcore_axis_name="s", num_cores=_NC, num_subcores=_NS)


_NB = 5
_NG = _NFULL // _NB - 2


@functools.partial(
    pl.kernel,
    out_type=jax.ShapeDtypeStruct((_NC, _N, _D), jnp.float32),
    mesh=_mesh,
    scratch_types=[
        pltpu.VMEM((_EPW,), jnp.int32),
        pltpu.VMEM((_EPW,), jnp.int32),
        pltpu.VMEM((_CH,), jnp.int32),
        pltpu.VMEM((_CH,), jnp.int32),
        pltpu.VMEM((_CH,), jnp.int32),
        pltpu.VMEM((_CH,), jnp.int32),
        pltpu.VMEM((_CH,), jnp.int32),
        pltpu.VMEM((_CH, _D), jnp.float32),
        pltpu.VMEM((_CH, _D), jnp.float32),
        pltpu.VMEM((_CH, _D), jnp.float32),
        pltpu.VMEM((_CH, _D), jnp.float32),
        pltpu.VMEM((_CH, _D), jnp.float32),
        pltpu.VMEM_SHARED((_N, _D), jnp.float32),
        pltpu.SemaphoreType.DMA,
        pltpu.SemaphoreType.DMA,
        pltpu.SemaphoreType.DMA,
        pltpu.SemaphoreType.DMA,
        pltpu.SemaphoreType.DMA,
        pltpu.SemaphoreType.DMA,
        pltpu.SemaphoreType.DMA,
        pltpu.SemaphoreType.DMA,
        pltpu.SemaphoreType.DMA,
        pltpu.SemaphoreType.DMA,
    ],
)
def _sc_scatter(z_hbm, src_hbm, dst_hbm, zeros_hbm, out_hbm,
                src_all, dst_all, d0, d1, d2, d3, d4, r0, r1, r2, r3, r4, acc,
                g0, g1, g2, g3, g4, c0, c1, c2, c3, c4):
    dch = (d0, d1, d2, d3, d4)
    rows = (r0, r1, r2, r3, r4)
    gsem = (g0, g1, g2, g3, g4)
    ssem = (c0, c1, c2, c3, c4)
    cid = lax.axis_index("c")
    sid = lax.axis_index("s")
    wid = cid * _NS + sid
    ebase = wid * _EPW

    pltpu.async_copy(src_hbm.at[pl.ds(ebase, _EPW)], src_all, g0)
    pltpu.async_copy(dst_hbm.at[pl.ds(ebase, _EPW)], dst_all, g1)
    pltpu.async_copy(zeros_hbm, acc.at[pl.ds(sid * _WR, _WR)], g2)

    @pl.when(sid == 0)
    def _():
        pltpu.sync_copy(zeros_hbm.at[pl.ds(0, _WREM)],
                        acc.at[pl.ds(_NS * _WR, _WREM)])
    pltpu.make_async_copy(src_hbm.at[pl.ds(0, _EPW)], src_all, g0).wait()
    pltpu.make_async_copy(dst_hbm.at[pl.ds(0, _EPW)], dst_all, g1).wait()
    pltpu.make_async_copy(zeros_hbm, acc.at[pl.ds(0, _WR)], g2).wait()
    plsc.subcore_barrier()

    def issue_gather(j, b):
        pltpu.async_copy(
            z_hbm.at[src_all.at[pl.ds(j * _CH, _CH)]], rows[b], gsem[b])

    def wait_gather(b):
        pltpu.make_async_copy(z_hbm.at[pl.ds(0, _CH)], rows[b], gsem[b]).wait()

    def issue_scatter(b):
        pltpu.async_copy(rows[b], acc.at[dch[b]], ssem[b], add=True)

    def wait_scatter(b):
        pltpu.make_async_copy(rows[b], acc.at[dch[b]], ssem[b]).wait()

    def fill_dst(j, b):
        dch[b][pl.ds(0, 16)] = dst_all[pl.ds(j * _CH, 16)]
        dch[b][pl.ds(16, 16)] = dst_all[pl.ds(j * _CH + 16, 16)]
        dch[b][pl.ds(24, 16)] = dst_all[pl.ds(j * _CH + 24, 16)]

    def pipe_step(j, b, do_wait, do_issue):
        sslot = (b + 3) % _NB
        if do_wait:
            wait_scatter(sslot)
        if do_issue:
            issue_gather(j + 3, sslot)
        fill_dst(j, b)
        wait_gather(b)
        issue_scatter(b)

    issue_gather(0, 0)
    issue_gather(1, 1)
    issue_gather(2, 2)
    pipe_step(0, 0, False, True)
    pipe_step(1, 1, False, True)
    pipe_step(2, 2, True, True)
    pipe_step(3, 3, True, True)
    pipe_step(4, 4, True, True)

    def group(g, _):
        for b in range(_NB):
            pipe_step(g * _NB + b, b, True, True)
        return 0
    lax.fori_loop(1, _NG + 1, group, 0)

    pipe_step(_NFULL - 5, 0, True, True)
    pipe_step(_NFULL - 4, 1, True, True)
    pipe_step(_NFULL - 3, 2, True, False)
    pipe_step(_NFULL - 2, 3, True, False)
    pipe_step(_NFULL - 1, 4, True, False)
    wait_scatter(3)
    wait_scatter(4)
    plsc.subcore_barrier()

    r0 = sid * _WR
    pltpu.sync_copy(acc.at[pl.ds(r0, _WR)], out_hbm.at[cid, pl.ds(r0, _WR)])

    @pl.when(sid == 0)
    def _():
        rr = _NS * _WR
        pltpu.sync_copy(acc.at[pl.ds(rr, _WREM)],
                        out_hbm.at[cid, pl.ds(rr, _WREM)])


_RB = 1000
_NGRID = _N // _RB


def _layer_body(z_ref, agg_ref, w1_ref, b1_ref, w2_ref, b2_ref, g_ref, be_ref,
                am_ref, ao_ref, o_ref, p_scr, s_scr, q_scr):
    i = pl.program_id(0)

    @pl.when(i < _NGRID)
    def _():
        h = z_ref[...] + agg_ref[0] + agg_ref[1]
        h = (jnp.dot(h, w1_ref[...], preferred_element_type=jnp.float32)
             + b1_ref[...])
        am = am_ref[0]
        h = jnp.where(h >= 0, h, am * h)
        y = (jnp.dot(h, w2_ref[...], preferred_element_type=jnp.float32)
             + b2_ref[...])
        ao = ao_ref[0]
        p = jnp.where(y >= 0, y, ao * y)
        p_scr[pl.ds(i * _RB, _RB), :] = p
        ps = jnp.sum(p.reshape(_RB // 8, 8, _D), axis=0)
        pq = jnp.sum((p * p).reshape(_RB // 8, 8, _D), axis=0)

        @pl.when(i == 0)
        def _():
            s_scr[...] = jnp.zeros((8, _D), jnp.float32)
            q_scr[...] = jnp.zeros((8, _D), jnp.float32)

        s_scr[...] += ps
        q_scr[...] += pq

    @pl.when(i >= _NGRID)
    def _():
        k = i - _NGRID
        s = jnp.sum(s_scr[...], axis=0, keepdims=True)
        q = jnp.sum(q_scr[...], axis=0, keepdims=True)
        mu = s / _N
        var = q / _N - mu * mu
        inv = 1.0 / jnp.sqrt(var + 1e-5)
        p = p_scr[pl.ds(k * _RB, _RB), :]
        o_ref[...] = g_ref[...] * (p - mu) * inv + be_ref[...]


def _layer_tc(z, agg, W1, b1, W2, b2, g, be, am, ao):
    clamp = lambda i: (jnp.minimum(i, _NGRID - 1), 0)
    fixed = lambda i: (0, 0)
    return pl.pallas_call(
        _layer_body,
        grid=(2 * _NGRID,),
        in_specs=[
            pl.BlockSpec((_RB, _D), clamp),
            pl.BlockSpec((_NC, _RB, _D), lambda i: (0, jnp.minimum(i, _NGRID - 1), 0)),
            pl.BlockSpec((_D, _D), fixed),
            pl.BlockSpec((1, _D), fixed),
            pl.BlockSpec((_D, _D), fixed),
            pl.BlockSpec((1, _D), fixed),
            pl.BlockSpec((1, _D), fixed),
            pl.BlockSpec((1, _D), fixed),
            pl.BlockSpec(memory_space=pltpu.SMEM),
            pl.BlockSpec(memory_space=pltpu.SMEM),
        ],
        out_specs=pl.BlockSpec(
            (_RB, _D), lambda i: (jnp.where(i < _NGRID, 0, i - _NGRID), 0)),
        out_shape=jax.ShapeDtypeStruct((_N, _D), jnp.float32),
        scratch_shapes=[
            pltpu.VMEM((_N, _D), jnp.float32),
            pltpu.VMEM((8, _D), jnp.float32),
            pltpu.VMEM((8, _D), jnp.float32),
        ],
    )(z, agg, W1, b1, W2, b2, g, be, am, ao)


def kernel(x, edge_index, batch, W1_0, b1_0, a1_0, W2_0, b2_0,
           W1_1, b1_1, a1_1, W2_1, b2_1, a_out, g0, be0, g1, be1):
    src = edge_index[0]
    dst = edge_index[1]
    zeros128 = jnp.zeros((_WR, _D), jnp.float32)
    ao = a_out.reshape(1)

    def layer(z, W1, b1, am, W2, b2, g, be):
        agg = _sc_scatter(z, src, dst, zeros128)
        return _layer_tc(z, agg, W1, b1.reshape(1, _D), W2, b2.reshape(1, _D),
                         g.reshape(1, _D), be.reshape(1, _D), am.reshape(1), ao)

    z = layer(x, W1_0, b1_0, a1_0, W2_0, b2_0, g0, be0)
    z = layer(z, W1_1, b1_1, a1_1, W2_1, b2_1, g1, be1)
    return z


# speedup vs baseline: 12.7022x; 1.0421x over previous
import functools

import jax
import jax.numpy as jnp
from jax import lax
from jax.experimental import pallas as pl
from jax.experimental.pallas import tpu as pltpu
from jax.experimental.pallas import tpu_sc as plsc

_N, _D, _E = 10000, 128, 320000
_NC, _NS = 2, 16
_NW = _NC * _NS
_EPW = _E // _NW
_CH = 40
_NFULL = _EPW // _CH
_WR = 624
_WREM = _N - _NS * _WR

_mesh = plsc.VectorSubcoreMesh(
    core_axis_name="c", subcore_axis_name="s", num_cores=_NC, num_subcores=_NS)


_NB = 5
_NG = _NFULL // _NB - 2


@functools.partial(
    pl.kernel,
    out_type=jax.ShapeDtypeStruct((_NC, _N, _D), jnp.float32),
    mesh=_mesh,
    scratch_types=[
        pltpu.VMEM((_EPW,), jnp.int32),
        pltpu.VMEM((_EPW,), jnp.int32),
        pltpu.VMEM((_CH,), jnp.int32),
        pltpu.VMEM((_CH,), jnp.int32),
        pltpu.VMEM((_CH,), jnp.int32),
        pltpu.VMEM((_CH,), jnp.int32),
        pltpu.VMEM((_CH,), jnp.int32),
        pltpu.VMEM((_CH, _D), jnp.float32),
        pltpu.VMEM((_CH, _D), jnp.float32),
        pltpu.VMEM((_CH, _D), jnp.float32),
        pltpu.VMEM((_CH, _D), jnp.float32),
        pltpu.VMEM((_CH, _D), jnp.float32),
        pltpu.VMEM_SHARED((_N, _D), jnp.float32),
        pltpu.SemaphoreType.DMA,
        pltpu.SemaphoreType.DMA,
        pltpu.SemaphoreType.DMA,
        pltpu.SemaphoreType.DMA,
        pltpu.SemaphoreType.DMA,
        pltpu.SemaphoreType.DMA,
        pltpu.SemaphoreType.DMA,
        pltpu.SemaphoreType.DMA,
        pltpu.SemaphoreType.DMA,
        pltpu.SemaphoreType.DMA,
    ],
)
def _sc_scatter(z_hbm, src_hbm, dst_hbm, out_hbm,
                src_all, dst_all, d0, d1, d2, d3, d4, r0, r1, r2, r3, r4, acc,
                g0, g1, g2, g3, g4, c0, c1, c2, c3, c4):
    dch = (d0, d1, d2, d3, d4)
    rows = (r0, r1, r2, r3, r4)
    gsem = (g0, g1, g2, g3, g4)
    ssem = (c0, c1, c2, c3, c4)
    cid = lax.axis_index("c")
    sid = lax.axis_index("s")
    wid = cid * _NS + sid
    ebase = wid * _EPW

    pltpu.async_copy(src_hbm.at[pl.ds(ebase, _EPW)], src_all, g0)
    pltpu.async_copy(dst_hbm.at[pl.ds(ebase, _EPW)], dst_all, g1)

    def zrow(r, _):
        for c in range(_D // 16):
            r0[r, pl.ds(c * 16, 16)] = jnp.zeros((16,), jnp.float32)
        return 0
    lax.fori_loop(0, _CH, zrow, 0)

    for k in range(_WR // _CH):
        pltpu.async_copy(r0, acc.at[pl.ds(sid * _WR + k * _CH, _CH)], g2)
    _ZT = _WR - (_WR // _CH) * _CH
    pltpu.async_copy(r0.at[pl.ds(0, _ZT)],
                     acc.at[pl.ds(sid * _WR + _WR - _ZT, _ZT)], g2)

    @pl.when(sid == 0)
    def _():
        pltpu.sync_copy(r0.at[pl.ds(0, _WREM)],
                        acc.at[pl.ds(_NS * _WR, _WREM)])
    pltpu.make_async_copy(src_hbm.at[pl.ds(0, _EPW)], src_all, g0).wait()
    pltpu.make_async_copy(dst_hbm.at[pl.ds(0, _EPW)], dst_all, g1).wait()
    for k in range(_WR // _CH):
        pltpu.make_async_copy(r0, acc.at[pl.ds(0, _CH)], g2).wait()
    pltpu.make_async_copy(r0.at[pl.ds(0, _ZT)], acc.at[pl.ds(0, _ZT)], g2).wait()
    plsc.subcore_barrier()

    def issue_gather(j, b):
        pltpu.async_copy(
            z_hbm.at[src_all.at[pl.ds(j * _CH, _CH)]], rows[b], gsem[b])

    def wait_gather(b):
        pltpu.make_async_copy(z_hbm.at[pl.ds(0, _CH)], rows[b], gsem[b]).wait()

    def issue_scatter(b):
        pltpu.async_copy(rows[b], acc.at[dch[b]], ssem[b], add=True)

    def wait_scatter(b):
        pltpu.make_async_copy(rows[b], acc.at[dch[b]], ssem[b]).wait()

    def fill_dst(j, b):
        dch[b][pl.ds(0, 16)] = dst_all[pl.ds(j * _CH, 16)]
        dch[b][pl.ds(16, 16)] = dst_all[pl.ds(j * _CH + 16, 16)]
        dch[b][pl.ds(24, 16)] = dst_all[pl.ds(j * _CH + 24, 16)]

    def pipe_step(j, b, do_wait, do_issue):
        sslot = (b + 3) % _NB
        if do_wait:
            wait_scatter(sslot)
        if do_issue:
            issue_gather(j + 3, sslot)
        fill_dst(j, b)
        wait_gather(b)
        issue_scatter(b)

    issue_gather(0, 0)
    issue_gather(1, 1)
    issue_gather(2, 2)
    pipe_step(0, 0, False, True)
    pipe_step(1, 1, False, True)
    pipe_step(2, 2, True, True)
    pipe_step(3, 3, True, True)
    pipe_step(4, 4, True, True)

    def group(g, _):
        for b in range(_NB):
            pipe_step(g * _NB + b, b, True, True)
        return 0
    lax.fori_loop(1, _NG + 1, group, 0)

    pipe_step(_NFULL - 5, 0, True, True)
    pipe_step(_NFULL - 4, 1, True, True)
    pipe_step(_NFULL - 3, 2, True, False)
    pipe_step(_NFULL - 2, 3, True, False)
    pipe_step(_NFULL - 1, 4, True, False)
    wait_scatter(3)
    wait_scatter(4)
    plsc.subcore_barrier()

    r0 = sid * _WR
    pltpu.sync_copy(acc.at[pl.ds(r0, _WR)], out_hbm.at[cid, pl.ds(r0, _WR)])

    @pl.when(sid == 0)
    def _():
        rr = _NS * _WR
        pltpu.sync_copy(acc.at[pl.ds(rr, _WREM)],
                        out_hbm.at[cid, pl.ds(rr, _WREM)])


_RB = 1000
_NGRID = _N // _RB


def _layer_body(z_ref, agg_ref, w1_ref, b1_ref, w2_ref, b2_ref, g_ref, be_ref,
                am_ref, ao_ref, o_ref, p_scr, s_scr, q_scr):
    i = pl.program_id(0)

    @pl.when(i < _NGRID)
    def _():
        h = z_ref[...] + agg_ref[0] + agg_ref[1]
        h = (jnp.dot(h, w1_ref[...], preferred_element_type=jnp.float32)
             + b1_ref[...])
        am = am_ref[0]
        h = jnp.where(h >= 0, h, am * h)
        y = (jnp.dot(h, w2_ref[...], preferred_element_type=jnp.float32)
             + b2_ref[...])
        ao = ao_ref[0]
        p = jnp.where(y >= 0, y, ao * y)
        p_scr[pl.ds(i * _RB, _RB), :] = p
        ps = jnp.sum(p.reshape(_RB // 8, 8, _D), axis=0)
        pq = jnp.sum((p * p).reshape(_RB // 8, 8, _D), axis=0)

        @pl.when(i == 0)
        def _():
            s_scr[...] = jnp.zeros((8, _D), jnp.float32)
            q_scr[...] = jnp.zeros((8, _D), jnp.float32)

        s_scr[...] += ps
        q_scr[...] += pq

    @pl.when(i >= _NGRID)
    def _():
        k = i - _NGRID
        s = jnp.sum(s_scr[...], axis=0, keepdims=True)
        q = jnp.sum(q_scr[...], axis=0, keepdims=True)
        mu = s / _N
        var = q / _N - mu * mu
        inv = 1.0 / jnp.sqrt(var + 1e-5)
        p = p_scr[pl.ds(k * _RB, _RB), :]
        o_ref[...] = g_ref[...] * (p - mu) * inv + be_ref[...]


def _layer_tc(z, agg, W1, b1, W2, b2, g, be, am, ao):
    clamp = lambda i: (jnp.minimum(i, _NGRID - 1), 0)
    fixed = lambda i: (0, 0)
    return pl.pallas_call(
        _layer_body,
        grid=(2 * _NGRID,),
        in_specs=[
            pl.BlockSpec((_RB, _D), clamp),
            pl.BlockSpec((_NC, _RB, _D), lambda i: (0, jnp.minimum(i, _NGRID - 1), 0)),
            pl.BlockSpec((_D, _D), fixed),
            pl.BlockSpec((1, _D), fixed),
            pl.BlockSpec((_D, _D), fixed),
            pl.BlockSpec((1, _D), fixed),
            pl.BlockSpec((1, _D), fixed),
            pl.BlockSpec((1, _D), fixed),
            pl.BlockSpec(memory_space=pltpu.SMEM),
            pl.BlockSpec(memory_space=pltpu.SMEM),
        ],
        out_specs=pl.BlockSpec(
            (_RB, _D), lambda i: (jnp.where(i < _NGRID, 0, i - _NGRID), 0)),
        out_shape=jax.ShapeDtypeStruct((_N, _D), jnp.float32),
        scratch_shapes=[
            pltpu.VMEM((_N, _D), jnp.float32),
            pltpu.VMEM((8, _D), jnp.float32),
            pltpu.VMEM((8, _D), jnp.float32),
        ],
    )(z, agg, W1, b1, W2, b2, g, be, am, ao)


def kernel(x, edge_index, batch, W1_0, b1_0, a1_0, W2_0, b2_0,
           W1_1, b1_1, a1_1, W2_1, b2_1, a_out, g0, be0, g1, be1):
    src = edge_index[0]
    dst = edge_index[1]
    ao = a_out.reshape(1)

    def layer(z, W1, b1, am, W2, b2, g, be):
        agg = _sc_scatter(z, src, dst)
        return _layer_tc(z, agg, W1, b1.reshape(1, _D), W2, b2.reshape(1, _D),
                         g.reshape(1, _D), be.reshape(1, _D), am.reshape(1), ao)

    z = layer(x, W1_0, b1_0, a1_0, W2_0, b2_0, g0, be0)
    z = layer(z, W1_1, b1_1, a1_1, W2_1, b2_1, g1, be1)
    return z


# speedup vs baseline: 12.8760x; 1.0137x over previous
import functools

import jax
import jax.numpy as jnp
from jax import lax
from jax.experimental import pallas as pl
from jax.experimental.pallas import tpu as pltpu
from jax.experimental.pallas import tpu_sc as plsc

_N, _D, _E = 10000, 128, 320000
_NC, _NS = 2, 16
_NW = _NC * _NS
_EPW = _E // _NW
_CH = 80
_NFULL = _EPW // _CH
_WR = 624
_WREM = _N - _NS * _WR

_mesh = plsc.VectorSubcoreMesh(
    core_axis_name="c", subcore_axis_name="s", num_cores=_NC, num_subcores=_NS)


_NB = 3
_NG = _NFULL // _NB - 1


@functools.partial(
    pl.kernel,
    out_type=jax.ShapeDtypeStruct((_NC, _N, _D), jnp.float32),
    mesh=_mesh,
    scratch_types=[
        pltpu.VMEM((_EPW,), jnp.int32),
        pltpu.VMEM((_CH,), jnp.int32),
        pltpu.VMEM((_CH,), jnp.int32),
        pltpu.VMEM((_CH,), jnp.int32),
        pltpu.VMEM((_CH, _D), jnp.float32),
        pltpu.VMEM((_CH, _D), jnp.float32),
        pltpu.VMEM((_CH, _D), jnp.float32),
        pltpu.VMEM_SHARED((_N, _D), jnp.float32),
        pltpu.SemaphoreType.DMA,
        pltpu.SemaphoreType.DMA,
        pltpu.SemaphoreType.DMA,
        pltpu.SemaphoreType.DMA,
        pltpu.SemaphoreType.DMA,
        pltpu.SemaphoreType.DMA,
        pltpu.SemaphoreType.DMA,
        pltpu.SemaphoreType.DMA,
        pltpu.SemaphoreType.DMA,
    ],
)
def _sc_scatter(z_hbm, src_hbm, dst_hbm, out_hbm,
                src_all, d0, d1, d2, r0, r1, r2, acc,
                g0, g1, g2, c0, c1, c2, e0, e1, e2):
    dch = (d0, d1, d2)
    rows = (r0, r1, r2)
    gsem = (g0, g1, g2)
    ssem = (c0, c1, c2)
    dsem = (e0, e1, e2)
    cid = lax.axis_index("c")
    sid = lax.axis_index("s")
    wid = cid * _NS + sid
    ebase = wid * _EPW

    pltpu.async_copy(src_hbm.at[pl.ds(ebase, _EPW)], src_all, g0)

    def zrow(r, _):
        for c in range(_D // 16):
            r0[r, pl.ds(c * 16, 16)] = jnp.zeros((16,), jnp.float32)
        return 0
    lax.fori_loop(0, _CH, zrow, 0)

    for k in range(_WR // _CH):
        pltpu.async_copy(r0, acc.at[pl.ds(sid * _WR + k * _CH, _CH)], g2)
    _ZT = _WR - (_WR // _CH) * _CH
    pltpu.async_copy(r0.at[pl.ds(0, _ZT)],
                     acc.at[pl.ds(sid * _WR + _WR - _ZT, _ZT)], g2)

    @pl.when(sid == 0)
    def _():
        pltpu.sync_copy(r0.at[pl.ds(0, _WREM)],
                        acc.at[pl.ds(_NS * _WR, _WREM)])
    pltpu.make_async_copy(src_hbm.at[pl.ds(0, _EPW)], src_all, g0).wait()
    for k in range(_WR // _CH):
        pltpu.make_async_copy(r0, acc.at[pl.ds(0, _CH)], g2).wait()
    pltpu.make_async_copy(r0.at[pl.ds(0, _ZT)], acc.at[pl.ds(0, _ZT)], g2).wait()
    plsc.subcore_barrier()

    def issue_gather(j, b):
        pltpu.async_copy(
            z_hbm.at[src_all.at[pl.ds(j * _CH, _CH)]], rows[b], gsem[b])

    def wait_gather(b):
        pltpu.make_async_copy(z_hbm.at[pl.ds(0, _CH)], rows[b], gsem[b]).wait()

    def issue_scatter(b):
        pltpu.async_copy(rows[b], acc.at[dch[b]], ssem[b], add=True)

    def wait_scatter(b):
        pltpu.make_async_copy(rows[b], acc.at[dch[b]], ssem[b]).wait()

    def load_didx(j, b):
        pltpu.async_copy(
            dst_hbm.at[pl.ds(ebase + j * _CH, _CH)], dch[b], dsem[b])

    def wait_didx(b):
        pltpu.make_async_copy(
            dst_hbm.at[pl.ds(0, _CH)], dch[b], dsem[b]).wait()

    def pipe_step(j, b, do_wait, do_issue):
        sslot = (b + 2) % _NB
        if do_wait:
            wait_scatter(sslot)
        if do_issue:
            issue_gather(j + 2, sslot)
            load_didx(j + 2, sslot)
        wait_didx(b)
        wait_gather(b)
        issue_scatter(b)

    issue_gather(0, 0)
    issue_gather(1, 1)
    load_didx(0, 0)
    load_didx(1, 1)
    pipe_step(0, 0, False, True)
    pipe_step(1, 1, True, True)
    pipe_step(2, 2, True, True)

    def group(g, _):
        for b in range(_NB):
            pipe_step(g * _NB + b, b, True, True)
        return 0
    lax.fori_loop(1, _NG + 1, group, 0)

    pipe_step(_NFULL - 2, 0, True, False)
    pipe_step(_NFULL - 1, 1, True, False)
    wait_scatter(1)
    plsc.subcore_barrier()

    r0 = sid * _WR
    pltpu.sync_copy(acc.at[pl.ds(r0, _WR)], out_hbm.at[cid, pl.ds(r0, _WR)])

    @pl.when(sid == 0)
    def _():
        rr = _NS * _WR
        pltpu.sync_copy(acc.at[pl.ds(rr, _WREM)],
                        out_hbm.at[cid, pl.ds(rr, _WREM)])


_RB = 1000
_NGRID = _N // _RB


def _layer_body(z_ref, agg_ref, w1_ref, b1_ref, w2_ref, b2_ref, g_ref, be_ref,
                am_ref, ao_ref, o_ref, p_scr, s_scr, q_scr):
    i = pl.program_id(0)

    @pl.when(i < _NGRID)
    def _():
        h = z_ref[...] + agg_ref[0] + agg_ref[1]
        h = (jnp.dot(h, w1_ref[...], preferred_element_type=jnp.float32)
             + b1_ref[...])
        am = am_ref[0]
        h = jnp.where(h >= 0, h, am * h)
        y = (jnp.dot(h, w2_ref[...], preferred_element_type=jnp.float32)
             + b2_ref[...])
        ao = ao_ref[0]
        p = jnp.where(y >= 0, y, ao * y)
        p_scr[pl.ds(i * _RB, _RB), :] = p
        ps = jnp.sum(p.reshape(_RB // 8, 8, _D), axis=0)
        pq = jnp.sum((p * p).reshape(_RB // 8, 8, _D), axis=0)

        @pl.when(i == 0)
        def _():
            s_scr[...] = jnp.zeros((8, _D), jnp.float32)
            q_scr[...] = jnp.zeros((8, _D), jnp.float32)

        s_scr[...] += ps
        q_scr[...] += pq

    @pl.when(i >= _NGRID)
    def _():
        k = i - _NGRID
        s = jnp.sum(s_scr[...], axis=0, keepdims=True)
        q = jnp.sum(q_scr[...], axis=0, keepdims=True)
        mu = s / _N
        var = q / _N - mu * mu
        inv = 1.0 / jnp.sqrt(var + 1e-5)
        p = p_scr[pl.ds(k * _RB, _RB), :]
        o_ref[...] = g_ref[...] * (p - mu) * inv + be_ref[...]


def _layer_tc(z, agg, W1, b1, W2, b2, g, be, am, ao):
    clamp = lambda i: (jnp.minimum(i, _NGRID - 1), 0)
    fixed = lambda i: (0, 0)
    return pl.pallas_call(
        _layer_body,
        grid=(2 * _NGRID,),
        in_specs=[
            pl.BlockSpec((_RB, _D), clamp),
            pl.BlockSpec((_NC, _RB, _D), lambda i: (0, jnp.minimum(i, _NGRID - 1), 0)),
            pl.BlockSpec((_D, _D), fixed),
            pl.BlockSpec((1, _D), fixed),
            pl.BlockSpec((_D, _D), fixed),
            pl.BlockSpec((1, _D), fixed),
            pl.BlockSpec((1, _D), fixed),
            pl.BlockSpec((1, _D), fixed),
            pl.BlockSpec(memory_space=pltpu.SMEM),
            pl.BlockSpec(memory_space=pltpu.SMEM),
        ],
        out_specs=pl.BlockSpec(
            (_RB, _D), lambda i: (jnp.where(i < _NGRID, 0, i - _NGRID), 0)),
        out_shape=jax.ShapeDtypeStruct((_N, _D), jnp.float32),
        scratch_shapes=[
            pltpu.VMEM((_N, _D), jnp.float32),
            pltpu.VMEM((8, _D), jnp.float32),
            pltpu.VMEM((8, _D), jnp.float32),
        ],
    )(z, agg, W1, b1, W2, b2, g, be, am, ao)


def kernel(x, edge_index, batch, W1_0, b1_0, a1_0, W2_0, b2_0,
           W1_1, b1_1, a1_1, W2_1, b2_1, a_out, g0, be0, g1, be1):
    src = edge_index[0]
    dst = edge_index[1]
    ao = a_out.reshape(1)

    def layer(z, W1, b1, am, W2, b2, g, be):
        agg = _sc_scatter(z, src, dst)
        return _layer_tc(z, agg, W1, b1.reshape(1, _D), W2, b2.reshape(1, _D),
                         g.reshape(1, _D), be.reshape(1, _D), am.reshape(1), ao)

    z = layer(x, W1_0, b1_0, a1_0, W2_0, b2_0, g0, be0)
    z = layer(z, W1_1, b1_1, a1_1, W2_1, b2_1, g1, be1)
    return z


# speedup vs baseline: 13.3326x; 1.0355x over previous
import functools

import jax
import jax.numpy as jnp
from jax import lax
from jax.experimental import pallas as pl
from jax.experimental.pallas import tpu as pltpu
from jax.experimental.pallas import tpu_sc as plsc

_N, _D, _E = 10000, 128, 320000
_NC, _NS = 2, 16
_NW = _NC * _NS
_EPW = _E // _NW
_CH = 80
_NFULL = _EPW // _CH
_WR = 624
_WREM = _N - _NS * _WR

_mesh = plsc.VectorSubcoreMesh(
    core_axis_name="c", subcore_axis_name="s", num_cores=_NC, num_subcores=_NS)


_NB = 3
_NG = _NFULL // _NB - 1


@functools.partial(
    pl.kernel,
    out_type=jax.ShapeDtypeStruct((_NC, _N, _D), jnp.float32),
    mesh=_mesh,
    scratch_types=[
        pltpu.VMEM((_EPW,), jnp.int32),
        pltpu.VMEM((_CH,), jnp.int32),
        pltpu.VMEM((_CH,), jnp.int32),
        pltpu.VMEM((_CH,), jnp.int32),
        pltpu.VMEM((_CH, _D), jnp.float32),
        pltpu.VMEM((_CH, _D), jnp.float32),
        pltpu.VMEM((_CH, _D), jnp.float32),
        pltpu.VMEM_SHARED((_N, _D), jnp.float32),
        pltpu.SemaphoreType.DMA,
        pltpu.SemaphoreType.DMA,
        pltpu.SemaphoreType.DMA,
        pltpu.SemaphoreType.DMA,
        pltpu.SemaphoreType.DMA,
        pltpu.SemaphoreType.DMA,
        pltpu.SemaphoreType.DMA,
        pltpu.SemaphoreType.DMA,
        pltpu.SemaphoreType.DMA,
    ],
)
def _sc_scatter(z_hbm, src_hbm, dst_hbm, out_hbm,
                src_all, d0, d1, d2, r0, r1, r2, acc,
                g0, g1, g2, c0, c1, c2, e0, e1, e2):
    dch = (d0, d1, d2)
    rows = (r0, r1, r2)
    gsem = (g0, g1, g2)
    ssem = (c0, c1, c2)
    dsem = (e0, e1, e2)
    cid = lax.axis_index("c")
    sid = lax.axis_index("s")
    wid = cid * _NS + sid
    ebase = wid * _EPW

    pltpu.async_copy(src_hbm.at[pl.ds(ebase, _EPW)], src_all, g0)

    def zrow(r, _):
        for c in range(_D // 16):
            r0[r, pl.ds(c * 16, 16)] = jnp.zeros((16,), jnp.float32)
        return 0
    lax.fori_loop(0, _CH, zrow, 0)

    for k in range(_WR // _CH):
        pltpu.async_copy(r0, acc.at[pl.ds(sid * _WR + k * _CH, _CH)], g2)
    _ZT = _WR - (_WR // _CH) * _CH
    pltpu.async_copy(r0.at[pl.ds(0, _ZT)],
                     acc.at[pl.ds(sid * _WR + _WR - _ZT, _ZT)], g2)

    @pl.when(sid == 0)
    def _():
        pltpu.sync_copy(r0.at[pl.ds(0, _WREM)],
                        acc.at[pl.ds(_NS * _WR, _WREM)])
    pltpu.make_async_copy(src_hbm.at[pl.ds(0, _EPW)], src_all, g0).wait()
    for k in range(_WR // _CH):
        pltpu.make_async_copy(r0, acc.at[pl.ds(0, _CH)], g2).wait()
    pltpu.make_async_copy(r0.at[pl.ds(0, _ZT)], acc.at[pl.ds(0, _ZT)], g2).wait()
    plsc.subcore_barrier()

    def issue_gather(j, b):
        pltpu.async_copy(
            z_hbm.at[src_all.at[pl.ds(j * _CH, _CH)]], rows[b], gsem[b])

    def wait_gather(b):
        pltpu.make_async_copy(z_hbm.at[pl.ds(0, _CH)], rows[b], gsem[b]).wait()

    def issue_scatter(b):
        pltpu.async_copy(rows[b], acc.at[dch[b]], ssem[b], add=True)

    def wait_scatter(b):
        pltpu.make_async_copy(rows[b], acc.at[dch[b]], ssem[b]).wait()

    def load_didx(j, b):
        pltpu.async_copy(
            dst_hbm.at[pl.ds(ebase + j * _CH, _CH)], dch[b], dsem[b])

    def wait_didx(b):
        pltpu.make_async_copy(
            dst_hbm.at[pl.ds(0, _CH)], dch[b], dsem[b]).wait()

    def pipe_step(j, b, do_wait, do_issue):
        sslot = (b + 2) % _NB
        if do_wait:
            wait_scatter(sslot)
        if do_issue:
            issue_gather(j + 2, sslot)
            load_didx(j + 2, sslot)
        wait_didx(b)
        wait_gather(b)
        issue_scatter(b)

    issue_gather(0, 0)
    issue_gather(1, 1)
    load_didx(0, 0)
    load_didx(1, 1)
    pipe_step(0, 0, False, True)
    pipe_step(1, 1, True, True)
    pipe_step(2, 2, True, True)

    def group(g, _):
        for b in range(_NB):
            pipe_step(g * _NB + b, b, True, True)
        return 0
    lax.fori_loop(1, _NG + 1, group, 0)

    pipe_step(_NFULL - 2, 0, True, False)
    pipe_step(_NFULL - 1, 1, True, False)
    wait_scatter(1)
    plsc.subcore_barrier()

    r0 = sid * _WR
    pltpu.sync_copy(acc.at[pl.ds(r0, _WR)], out_hbm.at[cid, pl.ds(r0, _WR)])

    @pl.when(sid == 0)
    def _():
        rr = _NS * _WR
        pltpu.sync_copy(acc.at[pl.ds(rr, _WREM)],
                        out_hbm.at[cid, pl.ds(rr, _WREM)])


_RB = 2000
_NGRID = _N // _RB


def _layer_body(z_ref, agg_ref, w1_ref, b1_ref, w2_ref, b2_ref, g_ref, be_ref,
                am_ref, ao_ref, o_ref, p_scr, s_scr, q_scr):
    i = pl.program_id(0)

    @pl.when(i < _NGRID)
    def _():
        h = z_ref[...] + agg_ref[0] + agg_ref[1]
        h = (jnp.dot(h, w1_ref[...], preferred_element_type=jnp.float32)
             + b1_ref[...])
        am = am_ref[0]
        h = jnp.where(h >= 0, h, am * h)
        y = (jnp.dot(h, w2_ref[...], preferred_element_type=jnp.float32)
             + b2_ref[...])
        ao = ao_ref[0]
        p = jnp.where(y >= 0, y, ao * y)
        p_scr[pl.ds(i * _RB, _RB), :] = p
        ps = jnp.sum(p.reshape(_RB // 8, 8, _D), axis=0)
        pq = jnp.sum((p * p).reshape(_RB // 8, 8, _D), axis=0)

        @pl.when(i == 0)
        def _():
            s_scr[...] = jnp.zeros((8, _D), jnp.float32)
            q_scr[...] = jnp.zeros((8, _D), jnp.float32)

        s_scr[...] += ps
        q_scr[...] += pq

    @pl.when(i >= _NGRID)
    def _():
        k = i - _NGRID
        s = jnp.sum(s_scr[...], axis=0, keepdims=True)
        q = jnp.sum(q_scr[...], axis=0, keepdims=True)
        mu = s / _N
        var = q / _N - mu * mu
        inv = 1.0 / jnp.sqrt(var + 1e-5)
        p = p_scr[pl.ds(k * _RB, _RB), :]
        o_ref[...] = g_ref[...] * (p - mu) * inv + be_ref[...]


def _layer_tc(z, agg, W1, b1, W2, b2, g, be, am, ao):
    clamp = lambda i: (jnp.minimum(i, _NGRID - 1), 0)
    fixed = lambda i: (0, 0)
    return pl.pallas_call(
        _layer_body,
        grid=(2 * _NGRID,),
        in_specs=[
            pl.BlockSpec((_RB, _D), clamp),
            pl.BlockSpec((_NC, _RB, _D), lambda i: (0, jnp.minimum(i, _NGRID - 1), 0)),
            pl.BlockSpec((_D, _D), fixed),
            pl.BlockSpec((1, _D), fixed),
            pl.BlockSpec((_D, _D), fixed),
            pl.BlockSpec((1, _D), fixed),
            pl.BlockSpec((1, _D), fixed),
            pl.BlockSpec((1, _D), fixed),
            pl.BlockSpec(memory_space=pltpu.SMEM),
            pl.BlockSpec(memory_space=pltpu.SMEM),
        ],
        out_specs=pl.BlockSpec(
            (_RB, _D), lambda i: (jnp.where(i < _NGRID, 0, i - _NGRID), 0)),
        out_shape=jax.ShapeDtypeStruct((_N, _D), jnp.float32),
        scratch_shapes=[
            pltpu.VMEM((_N, _D), jnp.float32),
            pltpu.VMEM((8, _D), jnp.float32),
            pltpu.VMEM((8, _D), jnp.float32),
        ],
    )(z, agg, W1, b1, W2, b2, g, be, am, ao)


def kernel(x, edge_index, batch, W1_0, b1_0, a1_0, W2_0, b2_0,
           W1_1, b1_1, a1_1, W2_1, b2_1, a_out, g0, be0, g1, be1):
    src = edge_index[0]
    dst = edge_index[1]
    ao = a_out.reshape(1)

    def layer(z, W1, b1, am, W2, b2, g, be):
        agg = _sc_scatter(z, src, dst)
        return _layer_tc(z, agg, W1, b1.reshape(1, _D), W2, b2.reshape(1, _D),
                         g.reshape(1, _D), be.reshape(1, _D), am.reshape(1), ao)

    z = layer(x, W1_0, b1_0, a1_0, W2_0, b2_0, g0, be0)
    z = layer(z, W1_1, b1_1, a1_1, W2_1, b2_1, g1, be1)
    return z


# speedup vs baseline: 13.3639x; 1.0023x over previous
import functools

import jax
import jax.numpy as jnp
from jax import lax
from jax.experimental import pallas as pl
from jax.experimental.pallas import tpu as pltpu
from jax.experimental.pallas import tpu_sc as plsc

_N, _D, _E = 10000, 128, 320000
_NC, _NS = 2, 16
_NW = _NC * _NS
_EPW = _E // _NW
_CH = 80
_NFULL = _EPW // _CH
_WR = 624
_WREM = _N - _NS * _WR

_mesh = plsc.VectorSubcoreMesh(
    core_axis_name="c", subcore_axis_name="s", num_cores=_NC, num_subcores=_NS)


_NB = 3
_NG = _NFULL // _NB - 1


@functools.partial(
    pl.kernel,
    out_type=jax.ShapeDtypeStruct((_NC, _N, _D), jnp.float32),
    mesh=_mesh,
    scratch_types=[
        pltpu.VMEM((_EPW,), jnp.int32),
        pltpu.VMEM((_CH,), jnp.int32),
        pltpu.VMEM((_CH,), jnp.int32),
        pltpu.VMEM((_CH,), jnp.int32),
        pltpu.VMEM((_CH, _D), jnp.float32),
        pltpu.VMEM((_CH, _D), jnp.float32),
        pltpu.VMEM((_CH, _D), jnp.float32),
        pltpu.VMEM_SHARED((_N, _D), jnp.float32),
        pltpu.SemaphoreType.DMA,
        pltpu.SemaphoreType.DMA,
        pltpu.SemaphoreType.DMA,
        pltpu.SemaphoreType.DMA,
        pltpu.SemaphoreType.DMA,
        pltpu.SemaphoreType.DMA,
        pltpu.SemaphoreType.DMA,
        pltpu.SemaphoreType.DMA,
        pltpu.SemaphoreType.DMA,
    ],
)
def _sc_scatter(z_hbm, src_hbm, dst_hbm, out_hbm,
                src_all, d0, d1, d2, r0, r1, r2, acc,
                g0, g1, g2, c0, c1, c2, e0, e1, e2):
    dch = (d0, d1, d2)
    rows = (r0, r1, r2)
    gsem = (g0, g1, g2)
    ssem = (c0, c1, c2)
    dsem = (e0, e1, e2)
    cid = lax.axis_index("c")
    sid = lax.axis_index("s")
    wid = cid * _NS + sid
    ebase = wid * _EPW

    pltpu.async_copy(src_hbm.at[pl.ds(ebase, _EPW)], src_all, g0)

    def zrow(r, _):
        for c in range(_D // 16):
            r0[r, pl.ds(c * 16, 16)] = jnp.zeros((16,), jnp.float32)
        return 0
    lax.fori_loop(0, _CH, zrow, 0)

    for k in range(_WR // _CH):
        pltpu.async_copy(r0, acc.at[pl.ds(sid * _WR + k * _CH, _CH)], g2)
    _ZT = _WR - (_WR // _CH) * _CH
    pltpu.async_copy(r0.at[pl.ds(0, _ZT)],
                     acc.at[pl.ds(sid * _WR + _WR - _ZT, _ZT)], g2)

    @pl.when(sid == 0)
    def _():
        pltpu.sync_copy(r0.at[pl.ds(0, _WREM)],
                        acc.at[pl.ds(_NS * _WR, _WREM)])
    pltpu.make_async_copy(src_hbm.at[pl.ds(0, _EPW)], src_all, g0).wait()
    for k in range(_WR // _CH):
        pltpu.make_async_copy(r0, acc.at[pl.ds(0, _CH)], g2).wait()
    pltpu.make_async_copy(r0.at[pl.ds(0, _ZT)], acc.at[pl.ds(0, _ZT)], g2).wait()
    plsc.subcore_barrier()

    def issue_gather(j, b):
        pltpu.async_copy(
            z_hbm.at[src_all.at[pl.ds(j * _CH, _CH)]], rows[b], gsem[b])

    def wait_gather(b):
        pltpu.make_async_copy(z_hbm.at[pl.ds(0, _CH)], rows[b], gsem[b]).wait()

    def issue_scatter(b):
        pltpu.async_copy(rows[b], acc.at[dch[b]], ssem[b], add=True)

    def wait_scatter(b):
        pltpu.make_async_copy(rows[b], acc.at[dch[b]], ssem[b]).wait()

    def load_didx(j, b):
        pltpu.async_copy(
            dst_hbm.at[pl.ds(ebase + j * _CH, _CH)], dch[b], dsem[b])

    def wait_didx(b):
        pltpu.make_async_copy(
            dst_hbm.at[pl.ds(0, _CH)], dch[b], dsem[b]).wait()

    def pipe_step(j, b, do_wait, do_issue):
        sslot = (b + 2) % _NB
        if do_wait:
            wait_scatter(sslot)
        if do_issue:
            issue_gather(j + 2, sslot)
            load_didx(j + 2, sslot)
        wait_didx(b)
        wait_gather(b)
        issue_scatter(b)

    issue_gather(0, 0)
    issue_gather(1, 1)
    load_didx(0, 0)
    load_didx(1, 1)
    pipe_step(0, 0, False, True)
    pipe_step(1, 1, True, True)
    pipe_step(2, 2, True, True)

    def group(g, _):
        for b in range(_NB):
            pipe_step(g * _NB + b, b, True, True)
        return 0
    lax.fori_loop(1, _NG + 1, group, 0)

    pipe_step(_NFULL - 2, 0, True, False)
    pipe_step(_NFULL - 1, 1, True, False)
    wait_scatter(1)
    plsc.subcore_barrier()

    row0 = sid * _WR
    pltpu.sync_copy(acc.at[pl.ds(row0, _WR)],
                    out_hbm.at[cid, pl.ds(row0, _WR)])

    @pl.when(sid == 0)
    def _():
        rr = _NS * _WR
        pltpu.sync_copy(acc.at[pl.ds(rr, _WREM)],
                        out_hbm.at[cid, pl.ds(rr, _WREM)])


_RB = 2000
_NGRID = _N // _RB


def _layer_body(z_ref, agg_ref, w1_ref, b1_ref, w2_ref, b2_ref, g_ref, be_ref,
                am_ref, ao_ref, o_ref, p_scr, s_scr, q_scr):
    i = pl.program_id(0)

    @pl.when(i < _NGRID)
    def _():
        h = z_ref[...] + agg_ref[0] + agg_ref[1]
        h = (jnp.dot(h, w1_ref[...], preferred_element_type=jnp.float32)
             + b1_ref[...])
        am = am_ref[0]
        h = jnp.where(h >= 0, h, am * h)
        y = (jnp.dot(h, w2_ref[...], preferred_element_type=jnp.float32)
             + b2_ref[...])
        ao = ao_ref[0]
        p = jnp.where(y >= 0, y, ao * y)
        p_scr[pl.ds(i * _RB, _RB), :] = p
        ps = jnp.sum(p.reshape(_RB // 8, 8, _D), axis=0)
        pq = jnp.sum((p * p).reshape(_RB // 8, 8, _D), axis=0)

        @pl.when(i == 0)
        def _():
            s_scr[...] = jnp.zeros((8, _D), jnp.float32)
            q_scr[...] = jnp.zeros((8, _D), jnp.float32)

        s_scr[...] += ps
        q_scr[...] += pq

    @pl.when(i >= _NGRID)
    def _():
        k = i - _NGRID
        s = jnp.sum(s_scr[...], axis=0, keepdims=True)
        q = jnp.sum(q_scr[...], axis=0, keepdims=True)
        mu = s / _N
        var = q / _N - mu * mu
        inv = 1.0 / jnp.sqrt(var + 1e-5)
        p = p_scr[pl.ds(k * _RB, _RB), :]
        o_ref[...] = g_ref[...] * (p - mu) * inv + be_ref[...]


def _layer_tc(z, agg, W1, b1, W2, b2, g, be, am, ao):
    clamp = lambda i: (jnp.minimum(i, _NGRID - 1), 0)
    fixed = lambda i: (0, 0)
    return pl.pallas_call(
        _layer_body,
        grid=(2 * _NGRID,),
        in_specs=[
            pl.BlockSpec((_RB, _D), clamp),
            pl.BlockSpec((_NC, _RB, _D), lambda i: (0, jnp.minimum(i, _NGRID - 1), 0)),
            pl.BlockSpec((_D, _D), fixed),
            pl.BlockSpec((1, _D), fixed),
            pl.BlockSpec((_D, _D), fixed),
            pl.BlockSpec((1, _D), fixed),
            pl.BlockSpec((1, _D), fixed),
            pl.BlockSpec((1, _D), fixed),
            pl.BlockSpec(memory_space=pltpu.SMEM),
            pl.BlockSpec(memory_space=pltpu.SMEM),
        ],
        out_specs=pl.BlockSpec(
            (_RB, _D), lambda i: (jnp.where(i < _NGRID, 0, i - _NGRID), 0)),
        out_shape=jax.ShapeDtypeStruct((_N, _D), jnp.float32),
        scratch_shapes=[
            pltpu.VMEM((_N, _D), jnp.float32),
            pltpu.VMEM((8, _D), jnp.float32),
            pltpu.VMEM((8, _D), jnp.float32),
        ],
    )(z, agg, W1, b1, W2, b2, g, be, am, ao)


def kernel(x, edge_index, batch, W1_0, b1_0, a1_0, W2_0, b2_0,
           W1_1, b1_1, a1_1, W2_1, b2_1, a_out, g0, be0, g1, be1):
    src = edge_index[0]
    dst = edge_index[1]
    ao = a_out.reshape(1)

    def layer(z, W1, b1, am, W2, b2, g, be):
        agg = _sc_scatter(z, src, dst)
        return _layer_tc(z, agg, W1, b1.reshape(1, _D), W2, b2.reshape(1, _D),
                         g.reshape(1, _D), be.reshape(1, _D), am.reshape(1), ao)

    z = layer(x, W1_0, b1_0, a1_0, W2_0, b2_0, g0, be0)
    z = layer(z, W1_1, b1_1, a1_1, W2_1, b2_1, g1, be1)
    return z


# speedup vs baseline: 13.5483x; 1.0138x over previous
import functools

import jax
import jax.numpy as jnp
from jax import lax
from jax.experimental import pallas as pl
from jax.experimental.pallas import tpu as pltpu
from jax.experimental.pallas import tpu_sc as plsc

_N, _D, _E = 10000, 128, 320000
_NC, _NS = 2, 16
_NW = _NC * _NS
_EPW = _E // _NW
_CH = 80
_NFULL = _EPW // _CH
_WR = 624
_WREM = _N - _NS * _WR

_mesh = plsc.VectorSubcoreMesh(
    core_axis_name="c", subcore_axis_name="s", num_cores=_NC, num_subcores=_NS)


_NB = 3
_NG = _NFULL // _NB - 1


@functools.partial(
    pl.kernel,
    out_type=jax.ShapeDtypeStruct((_NC, _N, _D), jnp.float32),
    mesh=_mesh,
    scratch_types=[
        pltpu.VMEM((_EPW,), jnp.int32),
        pltpu.VMEM((_CH,), jnp.int32),
        pltpu.VMEM((_CH,), jnp.int32),
        pltpu.VMEM((_CH,), jnp.int32),
        pltpu.VMEM((_CH, _D), jnp.float32),
        pltpu.VMEM((_CH, _D), jnp.float32),
        pltpu.VMEM((_CH, _D), jnp.float32),
        pltpu.VMEM_SHARED((_N, _D), jnp.float32),
        pltpu.SemaphoreType.DMA,
        pltpu.SemaphoreType.DMA,
        pltpu.SemaphoreType.DMA,
        pltpu.SemaphoreType.DMA,
        pltpu.SemaphoreType.DMA,
        pltpu.SemaphoreType.DMA,
        pltpu.SemaphoreType.DMA,
        pltpu.SemaphoreType.DMA,
        pltpu.SemaphoreType.DMA,
    ],
)
def _sc_scatter(z_hbm, src_hbm, dst_hbm, out_hbm,
                src_all, d0, d1, d2, r0, r1, r2, acc,
                g0, g1, g2, c0, c1, c2, e0, e1, e2):
    dch = (d0, d1, d2)
    rows = (r0, r1, r2)
    gsem = (g0, g1, g2)
    ssem = (c0, c1, c2)
    dsem = (e0, e1, e2)
    cid = lax.axis_index("c")
    sid = lax.axis_index("s")
    wid = cid * _NS + sid
    ebase = wid * _EPW

    pltpu.async_copy(src_hbm.at[pl.ds(ebase, _EPW)], src_all, g0)

    def zrow(r, _):
        for c in range(_D // 16):
            r0[r, pl.ds(c * 16, 16)] = jnp.zeros((16,), jnp.float32)
        return 0
    lax.fori_loop(0, _CH, zrow, 0)

    for k in range(_WR // _CH):
        pltpu.async_copy(r0, acc.at[pl.ds(sid * _WR + k * _CH, _CH)], g2)
    _ZT = _WR - (_WR // _CH) * _CH
    pltpu.async_copy(r0.at[pl.ds(0, _ZT)],
                     acc.at[pl.ds(sid * _WR + _WR - _ZT, _ZT)], g2)

    @pl.when(sid == 0)
    def _():
        pltpu.sync_copy(r0.at[pl.ds(0, _WREM)],
                        acc.at[pl.ds(_NS * _WR, _WREM)])
    pltpu.make_async_copy(src_hbm.at[pl.ds(0, _EPW)], src_all, g0).wait()
    for k in range(_WR // _CH):
        pltpu.make_async_copy(r0, acc.at[pl.ds(0, _CH)], g2).wait()
    pltpu.make_async_copy(r0.at[pl.ds(0, _ZT)], acc.at[pl.ds(0, _ZT)], g2).wait()
    plsc.subcore_barrier()

    def issue_gather(j, b):
        pltpu.async_copy(
            z_hbm.at[src_all.at[pl.ds(j * _CH, _CH)]], rows[b], gsem[b])

    def wait_gather(b):
        pltpu.make_async_copy(z_hbm.at[pl.ds(0, _CH)], rows[b], gsem[b]).wait()

    def issue_scatter(b):
        pltpu.async_copy(rows[b], acc.at[dch[b]], ssem[b], add=True)

    def wait_scatter(b):
        pltpu.make_async_copy(rows[b], acc.at[dch[b]], ssem[b]).wait()

    def load_didx(j, b):
        pltpu.async_copy(
            dst_hbm.at[pl.ds(ebase + j * _CH, _CH)], dch[b], dsem[b])

    def wait_didx(b):
        pltpu.make_async_copy(
            dst_hbm.at[pl.ds(0, _CH)], dch[b], dsem[b]).wait()

    def pipe_step(j, b, do_wait, do_issue):
        sslot = (b + 2) % _NB
        if do_wait:
            wait_scatter(sslot)
        if do_issue:
            issue_gather(j + 2, sslot)
            load_didx(j + 2, sslot)
        wait_didx(b)
        wait_gather(b)
        issue_scatter(b)

    issue_gather(0, 0)
    issue_gather(1, 1)
    load_didx(0, 0)
    load_didx(1, 1)
    pipe_step(0, 0, False, True)
    pipe_step(1, 1, True, True)
    pipe_step(2, 2, True, True)

    def group(g, _):
        for b in range(_NB):
            pipe_step(g * _NB + b, b, True, True)
        return 0
    lax.fori_loop(1, _NG + 1, group, 0)

    pipe_step(_NFULL - 2, 0, True, False)
    pipe_step(_NFULL - 1, 1, True, False)
    wait_scatter(1)
    plsc.subcore_barrier()

    row0 = sid * _WR
    pltpu.sync_copy(acc.at[pl.ds(row0, _WR)],
                    out_hbm.at[cid, pl.ds(row0, _WR)])

    @pl.when(sid == 0)
    def _():
        rr = _NS * _WR
        pltpu.sync_copy(acc.at[pl.ds(rr, _WREM)],
                        out_hbm.at[cid, pl.ds(rr, _WREM)])


_RB = 5000
_NGRID = _N // _RB


def _layer_body(z_ref, agg_ref, w1_ref, b1_ref, w2_ref, b2_ref, g_ref, be_ref,
                am_ref, ao_ref, o_ref, p_scr, s_scr, q_scr):
    i = pl.program_id(0)

    @pl.when(i < _NGRID)
    def _():
        h = z_ref[...] + agg_ref[0] + agg_ref[1]
        h = (jnp.dot(h, w1_ref[...], preferred_element_type=jnp.float32)
             + b1_ref[...])
        am = am_ref[0]
        h = jnp.where(h >= 0, h, am * h)
        y = (jnp.dot(h, w2_ref[...], preferred_element_type=jnp.float32)
             + b2_ref[...])
        ao = ao_ref[0]
        p = jnp.where(y >= 0, y, ao * y)
        p_scr[pl.ds(i * _RB, _RB), :] = p
        ps = jnp.sum(p.reshape(_RB // 8, 8, _D), axis=0)
        pq = jnp.sum((p * p).reshape(_RB // 8, 8, _D), axis=0)

        @pl.when(i == 0)
        def _():
            s_scr[...] = jnp.zeros((8, _D), jnp.float32)
            q_scr[...] = jnp.zeros((8, _D), jnp.float32)

        s_scr[...] += ps
        q_scr[...] += pq

    @pl.when(i >= _NGRID)
    def _():
        k = i - _NGRID
        s = jnp.sum(s_scr[...], axis=0, keepdims=True)
        q = jnp.sum(q_scr[...], axis=0, keepdims=True)
        mu = s / _N
        var = q / _N - mu * mu
        inv = 1.0 / jnp.sqrt(var + 1e-5)
        p = p_scr[pl.ds(k * _RB, _RB), :]
        o_ref[...] = g_ref[...] * (p - mu) * inv + be_ref[...]


def _layer_tc(z, agg, W1, b1, W2, b2, g, be, am, ao):
    clamp = lambda i: (jnp.minimum(i, _NGRID - 1), 0)
    fixed = lambda i: (0, 0)
    return pl.pallas_call(
        _layer_body,
        grid=(2 * _NGRID,),
        in_specs=[
            pl.BlockSpec((_RB, _D), clamp),
            pl.BlockSpec((_NC, _RB, _D), lambda i: (0, jnp.minimum(i, _NGRID - 1), 0)),
            pl.BlockSpec((_D, _D), fixed),
            pl.BlockSpec((1, _D), fixed),
            pl.BlockSpec((_D, _D), fixed),
            pl.BlockSpec((1, _D), fixed),
            pl.BlockSpec((1, _D), fixed),
            pl.BlockSpec((1, _D), fixed),
            pl.BlockSpec(memory_space=pltpu.SMEM),
            pl.BlockSpec(memory_space=pltpu.SMEM),
        ],
        out_specs=pl.BlockSpec(
            (_RB, _D), lambda i: (jnp.where(i < _NGRID, 0, i - _NGRID), 0)),
        out_shape=jax.ShapeDtypeStruct((_N, _D), jnp.float32),
        scratch_shapes=[
            pltpu.VMEM((_N, _D), jnp.float32),
            pltpu.VMEM((8, _D), jnp.float32),
            pltpu.VMEM((8, _D), jnp.float32),
        ],
    )(z, agg, W1, b1, W2, b2, g, be, am, ao)


def kernel(x, edge_index, batch, W1_0, b1_0, a1_0, W2_0, b2_0,
           W1_1, b1_1, a1_1, W2_1, b2_1, a_out, g0, be0, g1, be1):
    src = edge_index[0]
    dst = edge_index[1]
    ao = a_out.reshape(1)

    def layer(z, W1, b1, am, W2, b2, g, be):
        agg = _sc_scatter(z, src, dst)
        return _layer_tc(z, agg, W1, b1.reshape(1, _D), W2, b2.reshape(1, _D),
                         g.reshape(1, _D), be.reshape(1, _D), am.reshape(1), ao)

    z = layer(x, W1_0, b1_0, a1_0, W2_0, b2_0, g0, be0)
    z = layer(z, W1_1, b1_1, a1_1, W2_1, b2_1, g1, be1)
    return z
